# Initial kernel scaffold; baseline (speedup 1.0000x reference)
import jax
import jax.numpy as jnp
from jax.experimental import pallas as pl


def kernel(x, entity_emb, weight_velocity, W_second, b_second, W1, b1, W2, b2, adj_vals, eigen_val0, adj_rows, adj_cols, kg_src, kg_dst, batch_item_ids):
    raise NotImplementedError("write your pallas kernel here")


# speedup vs baseline: 3.2926x; 3.2926x over previous
import functools

import jax
import jax.numpy as jnp
from jax.experimental import pallas as pl
from jax.experimental.pallas import tpu as pltpu

_ALPHA = 2.5
_T = 2
_DROPOUT = 0.5


def _round_up(n, m):
    return ((n + m - 1) // m) * m


def _blk(n, target):
    b = min(n, target)
    while n % b:
        b -= 128
    return b


def _mm_kernel(a_ref, b_ref, o_ref, acc_ref, *, nk, trans_b):
    k = pl.program_id(2)

    @pl.when(k == 0)
    def _():
        acc_ref[...] = jnp.zeros_like(acc_ref)

    dn = (((1,), (1,)), ((), ())) if trans_b else (((1,), (0,)), ((), ()))
    acc_ref[...] += jax.lax.dot_general(
        a_ref[...], b_ref[...], dn, preferred_element_type=jnp.float32)

    @pl.when(k == nk - 1)
    def _():
        o_ref[...] = acc_ref[...].astype(o_ref.dtype)


def _matmul(a, b, *, trans_b=False, out_dtype=jnp.float32, bm=None, bn=None,
            bk=None):
    m, ka = a.shape
    if trans_b:
        n, kb = b.shape
    else:
        kb, n = b.shape
    assert ka == kb
    bm = bm or m
    bn = bn or _blk(n, 1024)
    bk = bk or _blk(ka, 1024)
    nk = ka // bk
    b_spec = (pl.BlockSpec((bn, bk), lambda i, j, k: (j, k)) if trans_b
              else pl.BlockSpec((bk, bn), lambda i, j, k: (k, j)))
    return pl.pallas_call(
        functools.partial(_mm_kernel, nk=nk, trans_b=trans_b),
        grid=(m // bm, n // bn, nk),
        in_specs=[pl.BlockSpec((bm, bk), lambda i, j, k: (i, k)), b_spec],
        out_specs=pl.BlockSpec((bm, bn), lambda i, j, k: (i, j)),
        out_shape=jax.ShapeDtypeStruct((m, n), out_dtype),
        scratch_shapes=[pltpu.VMEM((bm, bn), jnp.float32)],
        compiler_params=pltpu.CompilerParams(
            dimension_semantics=("parallel", "parallel", "arbitrary")),
    )(a, b)


def _kg_msg_kernel(src_ref, dst_ref, wv_ref, o_ref):
    d = src_ref[...]
    dst = dst_ref[...]
    v = jax.lax.dot_general(
        (dst - d), wv_ref[...], (((1,), (0,)), ((), ())),
        preferred_element_type=jnp.float32)
    o_ref[...] = jnp.tanh(v) * dst.astype(jnp.float32)


def _kg_messages(esrc, edst, wv, be=2000):
    e, d = esrc.shape
    be = min(be, e)
    assert e % be == 0
    return pl.pallas_call(
        _kg_msg_kernel,
        grid=(e // be,),
        in_specs=[
            pl.BlockSpec((be, d), lambda i: (i, 0)),
            pl.BlockSpec((be, d), lambda i: (i, 0)),
            pl.BlockSpec((d, d), lambda i: (0, 0)),
        ],
        out_specs=pl.BlockSpec((be, d), lambda i: (i, 0)),
        out_shape=jax.ShapeDtypeStruct((e, d), jnp.float32),
        compiler_params=pltpu.CompilerParams(
            dimension_semantics=("arbitrary",)),
    )(esrc, edst, wv)


def _assemble_kernel(x_ref, c_ref, ax_ref, ac_ref, secg_ref, ws_ref, bs_ref,
                     g1_ref, g2_ref, inv_eig_ref, o_ref):
    x = x_ref[...]
    sec = jax.lax.dot_general(
        secg_ref[...], ws_ref[...], (((1,), (0,)), ((), ())),
        preferred_element_type=jnp.float32) + bs_ref[...]
    z = x * g1_ref[...] + g2_ref[...] * ax_ref[...] + sec
    o_ref[0, :, :] = z.astype(o_ref.dtype)
    o_ref[1, :, :] = c_ref[...].astype(o_ref.dtype)
    o_ref[2, :, :] = (ac_ref[...] * inv_eig_ref[...]).astype(o_ref.dtype)


def _assemble(x, c_bf, s, secg, w_second, b_second2d, g1, g2, inv_eig):
    b, n = x.shape
    d = secg.shape[1]
    bn = _blk(n, 1024)
    return pl.pallas_call(
        _assemble_kernel,
        grid=(n // bn,),
        in_specs=[
            pl.BlockSpec((b, bn), lambda j: (0, j)),
            pl.BlockSpec((b, bn), lambda j: (1, j)),
            pl.BlockSpec((b, bn), lambda j: (0, j)),
            pl.BlockSpec((b, bn), lambda j: (1, j)),
            pl.BlockSpec((b, d), lambda j: (0, 0)),
            pl.BlockSpec((d, bn), lambda j: (0, j)),
            pl.BlockSpec((1, bn), lambda j: (0, j)),
            pl.BlockSpec((b, 1), lambda j: (0, 0)),
            pl.BlockSpec((b, 1), lambda j: (0, 0)),
            pl.BlockSpec((1, 1), lambda j: (0, 0)),
        ],
        out_specs=pl.BlockSpec((3, b, bn), lambda j: (0, 0, j)),
        out_shape=jax.ShapeDtypeStruct((3, b, n), jnp.bfloat16),
        compiler_params=pltpu.CompilerParams(
            dimension_semantics=("arbitrary",)),
    )(x, c_bf, s, s, secg, w_second, b_second2d, g1, g2, inv_eig)


def _mlp1_kernel(z_ref, w1_ref, o_ref, acc_ref, *, ns, nk):
    s = pl.program_id(0)
    k = pl.program_id(1)

    @pl.when((s == 0) & (k == 0))
    def _():
        acc_ref[...] = jnp.zeros_like(acc_ref)

    acc_ref[...] += jax.lax.dot_general(
        z_ref[0], w1_ref[0], (((1,), (0,)), ((), ())),
        preferred_element_type=jnp.float32)

    @pl.when((s == ns - 1) & (k == nk - 1))
    def _():
        o_ref[...] = acc_ref[...]


def _mlp1(zs, w1p):
    ns, b, n = zs.shape
    hid = w1p.shape[2]
    bk = _blk(n, 1024)
    nk = n // bk
    return pl.pallas_call(
        functools.partial(_mlp1_kernel, ns=ns, nk=nk),
        grid=(ns, nk),
        in_specs=[
            pl.BlockSpec((1, b, bk), lambda s, k: (s, 0, k)),
            pl.BlockSpec((1, bk, hid), lambda s, k: (s, k, 0)),
        ],
        out_specs=pl.BlockSpec((b, hid), lambda s, k: (0, 0)),
        out_shape=jax.ShapeDtypeStruct((b, hid), jnp.float32),
        scratch_shapes=[pltpu.VMEM((b, hid), jnp.float32)],
        compiler_params=pltpu.CompilerParams(
            dimension_semantics=("arbitrary", "arbitrary")),
    )(zs, w1p)


def _mlp2_kernel(h_ref, b1_ref, w1l_ref, tf_ref, w2_ref, b2_ref, x_ref,
                 o_ref, *, inv_b):
    j = pl.program_id(0)
    h = h_ref[...] + b1_ref[...] + tf_ref[...] * w1l_ref[...]
    hs = (h * jax.nn.sigmoid(h)).astype(jnp.bfloat16)
    pred = jax.lax.dot_general(
        hs, w2_ref[...], (((1,), (0,)), ((), ())),
        preferred_element_type=jnp.float32) + b2_ref[...]
    d = x_ref[...] - pred
    part = jnp.sum(d * d) * inv_b

    @pl.when(j == 0)
    def _():
        o_ref[...] = jnp.zeros_like(o_ref)

    o_ref[...] = o_ref[...] + part


def _mlp2_loss(h, b1_2d, w1_last, tf, w2, b2_2d, x):
    b, hid = h.shape
    n = w2.shape[1]
    bn = _blk(n, 1024)
    out = pl.pallas_call(
        functools.partial(_mlp2_kernel, inv_b=1.0 / b),
        grid=(n // bn,),
        in_specs=[
            pl.BlockSpec((b, hid), lambda j: (0, 0)),
            pl.BlockSpec((1, hid), lambda j: (0, 0)),
            pl.BlockSpec((1, hid), lambda j: (0, 0)),
            pl.BlockSpec((b, 1), lambda j: (0, 0)),
            pl.BlockSpec((hid, bn), lambda j: (0, j)),
            pl.BlockSpec((1, bn), lambda j: (0, j)),
            pl.BlockSpec((b, bn), lambda j: (0, j)),
        ],
        out_specs=pl.BlockSpec((1, 1), lambda j: (0, 0)),
        out_shape=jax.ShapeDtypeStruct((1, 1), jnp.float32),
        compiler_params=pltpu.CompilerParams(
            dimension_semantics=("arbitrary",)),
    )(h, b1_2d, w1_last, tf, w2, b2_2d, x)
    return out[0, 0]


def kernel(x, entity_emb, weight_velocity, W_second, b_second, W1, b1, W2, b2,
           adj_vals, eigen_val0, adj_rows, adj_cols, kg_src, kg_dst,
           batch_item_ids):
    b, n_items = x.shape
    n_ent, latdim = entity_emb.shape
    hid = W1.shape[1]
    np_ = _round_up(n_items, 128 if n_items < 1024 else 1024)
    pad = np_ - n_items

    key = jax.random.key(42)
    kt, kd = jax.random.split(key)
    t = jax.random.randint(kt, (b, 1), 1, _T + 1)
    tf = t.astype(jnp.float32)
    keep = jax.random.bernoulli(kd, 1.0 - _DROPOUT, x.shape)

    inv_keep = 1.0 / (1.0 - _DROPOUT)
    c = jnp.where(keep, inv_keep * x, 0.0)
    xp = jnp.pad(x, ((0, 0), (0, pad)))
    mbp = jnp.pad(jnp.concatenate([x, c], axis=0).astype(jnp.bfloat16),
                  ((0, 0), (0, pad)))

    a_dense = jnp.zeros((np_, np_), jnp.bfloat16)
    a_dense = a_dense.at[adj_rows, adj_cols].add(adj_vals.astype(jnp.bfloat16))

    p = _matmul(mbp, a_dense, trans_b=True, out_dtype=jnp.bfloat16)
    s = _matmul(p, a_dense, trans_b=False, out_dtype=jnp.float32)

    esrc = entity_emb[kg_src].astype(jnp.bfloat16)
    edst = entity_emb[kg_dst].astype(jnp.bfloat16)
    msg = _kg_messages(esrc, edst, weight_velocity.astype(jnp.bfloat16))
    sec = jax.ops.segment_sum(msg, kg_src, num_segments=n_ent)
    secg = sec[batch_item_ids]

    gamma = _ALPHA * tf / _T
    inv_eig = (1.0 / eigen_val0).reshape(1, 1)
    wsp = jnp.pad(W_second.astype(jnp.bfloat16), ((0, 0), (0, pad)))
    bsp = jnp.pad(b_second.reshape(1, n_items), ((0, 0), (0, pad)))
    zs = _assemble(xp, mbp, s, secg, wsp, bsp, 1.0 - gamma,
                   gamma * inv_eig[0, 0], inv_eig)

    w1p = jnp.pad(W1[:3 * n_items].reshape(3, n_items, hid).astype(jnp.bfloat16),
                  ((0, 0), (0, pad), (0, 0)))
    h = _mlp1(zs, w1p)

    w2p = jnp.pad(W2.astype(jnp.bfloat16), ((0, 0), (0, pad)))
    b2p = jnp.pad(b2.reshape(1, n_items), ((0, 0), (0, pad)))
    loss = _mlp2_loss(h, b1.reshape(1, -1), W1[-1:, :], tf, w2p, b2p, xp)
    return loss


# speedup vs baseline: 5.1309x; 1.5583x over previous
import functools

import jax
import jax.numpy as jnp
from jax.experimental import pallas as pl
from jax.experimental.pallas import tpu as pltpu

_ALPHA = 2.5
_T = 2
_DROPOUT = 0.5


def _round_up(n, m):
    return ((n + m - 1) // m) * m


def _blk(n, target):
    b = min(n, target)
    while n % b:
        b -= 128
    return b


def _mm_kernel(a_ref, b_ref, o_ref, acc_ref, *, nk, trans_b):
    k = pl.program_id(2)

    @pl.when(k == 0)
    def _():
        acc_ref[...] = jnp.zeros_like(acc_ref)

    dn = (((1,), (1,)), ((), ())) if trans_b else (((1,), (0,)), ((), ()))
    acc_ref[...] += jax.lax.dot_general(
        a_ref[...], b_ref[...], dn, preferred_element_type=jnp.float32)

    @pl.when(k == nk - 1)
    def _():
        o_ref[...] = acc_ref[...].astype(o_ref.dtype)


def _matmul(a, b, *, trans_b=False, out_dtype=jnp.float32, bm=None, bn=None,
            bk=None):
    m, ka = a.shape
    if trans_b:
        n, kb = b.shape
    else:
        kb, n = b.shape
    assert ka == kb
    bm = bm or m
    bn = bn or _blk(n, 1024)
    bk = bk or _blk(ka, 1024)
    nk = ka // bk
    b_spec = (pl.BlockSpec((bn, bk), lambda i, j, k: (j, k)) if trans_b
              else pl.BlockSpec((bk, bn), lambda i, j, k: (k, j)))
    return pl.pallas_call(
        functools.partial(_mm_kernel, nk=nk, trans_b=trans_b),
        grid=(m // bm, n // bn, nk),
        in_specs=[pl.BlockSpec((bm, bk), lambda i, j, k: (i, k)), b_spec],
        out_specs=pl.BlockSpec((bm, bn), lambda i, j, k: (i, j)),
        out_shape=jax.ShapeDtypeStruct((m, n), out_dtype),
        scratch_shapes=[pltpu.VMEM((bm, bn), jnp.float32)],
        compiler_params=pltpu.CompilerParams(
            dimension_semantics=("parallel", "parallel", "arbitrary")),
    )(a, b)


def _kg_msg_kernel(src_ref, dst_ref, wv_ref, o_ref):
    d = src_ref[...]
    dst = dst_ref[...]
    v = jax.lax.dot_general(
        (dst - d), wv_ref[...], (((1,), (0,)), ((), ())),
        preferred_element_type=jnp.float32)
    o_ref[...] = jnp.tanh(v) * dst.astype(jnp.float32)


def _kg_messages(esrc, edst, wv, be=2000):
    e, d = esrc.shape
    be = min(be, e)
    assert e % be == 0
    return pl.pallas_call(
        _kg_msg_kernel,
        grid=(e // be,),
        in_specs=[
            pl.BlockSpec((be, d), lambda i: (i, 0)),
            pl.BlockSpec((be, d), lambda i: (i, 0)),
            pl.BlockSpec((d, d), lambda i: (0, 0)),
        ],
        out_specs=pl.BlockSpec((be, d), lambda i: (i, 0)),
        out_shape=jax.ShapeDtypeStruct((e, d), jnp.float32),
        compiler_params=pltpu.CompilerParams(
            dimension_semantics=("arbitrary",)),
    )(esrc, edst, wv)


def _assemble_kernel(x_ref, c_ref, ax_ref, ac_ref, secg_ref, ws_ref, bs_ref,
                     g1_ref, g2_ref, inv_eig_ref, o_ref):
    x = x_ref[...]
    sec = jax.lax.dot_general(
        secg_ref[...], ws_ref[...], (((1,), (0,)), ((), ())),
        preferred_element_type=jnp.float32) + bs_ref[...]
    z = x * g1_ref[...] + g2_ref[...] * ax_ref[...] + sec
    o_ref[0, :, :] = z.astype(o_ref.dtype)
    o_ref[1, :, :] = c_ref[...].astype(o_ref.dtype)
    o_ref[2, :, :] = (ac_ref[...] * inv_eig_ref[...]).astype(o_ref.dtype)


def _assemble(x, c_bf, s, secg, w_second, b_second2d, g1, g2, inv_eig):
    b, n = x.shape
    d = secg.shape[1]
    bn = _blk(n, 1024)
    return pl.pallas_call(
        _assemble_kernel,
        grid=(n // bn,),
        in_specs=[
            pl.BlockSpec((b, bn), lambda j: (0, j)),
            pl.BlockSpec((b, bn), lambda j: (1, j)),
            pl.BlockSpec((b, bn), lambda j: (0, j)),
            pl.BlockSpec((b, bn), lambda j: (1, j)),
            pl.BlockSpec((b, d), lambda j: (0, 0)),
            pl.BlockSpec((d, bn), lambda j: (0, j)),
            pl.BlockSpec((1, bn), lambda j: (0, j)),
            pl.BlockSpec((b, 1), lambda j: (0, 0)),
            pl.BlockSpec((b, 1), lambda j: (0, 0)),
            pl.BlockSpec((1, 1), lambda j: (0, 0)),
        ],
        out_specs=pl.BlockSpec((3, b, bn), lambda j: (0, 0, j)),
        out_shape=jax.ShapeDtypeStruct((3, b, n), jnp.bfloat16),
        compiler_params=pltpu.CompilerParams(
            dimension_semantics=("arbitrary",)),
    )(x, c_bf, s, s, secg, w_second, b_second2d, g1, g2, inv_eig)


def _mlp1_kernel(z_ref, w1_ref, o_ref, acc_ref, *, ns, nk):
    s = pl.program_id(0)
    k = pl.program_id(1)

    @pl.when((s == 0) & (k == 0))
    def _():
        acc_ref[...] = jnp.zeros_like(acc_ref)

    acc_ref[...] += jax.lax.dot_general(
        z_ref[0], w1_ref[0], (((1,), (0,)), ((), ())),
        preferred_element_type=jnp.float32)

    @pl.when((s == ns - 1) & (k == nk - 1))
    def _():
        o_ref[...] = acc_ref[...]


def _mlp1(zs, w1p):
    ns, b, n = zs.shape
    hid = w1p.shape[2]
    bk = _blk(n, 1024)
    nk = n // bk
    return pl.pallas_call(
        functools.partial(_mlp1_kernel, ns=ns, nk=nk),
        grid=(ns, nk),
        in_specs=[
            pl.BlockSpec((1, b, bk), lambda s, k: (s, 0, k)),
            pl.BlockSpec((1, bk, hid), lambda s, k: (s, k, 0)),
        ],
        out_specs=pl.BlockSpec((b, hid), lambda s, k: (0, 0)),
        out_shape=jax.ShapeDtypeStruct((b, hid), jnp.float32),
        scratch_shapes=[pltpu.VMEM((b, hid), jnp.float32)],
        compiler_params=pltpu.CompilerParams(
            dimension_semantics=("arbitrary", "arbitrary")),
    )(zs, w1p)


def _mlp2_kernel(h_ref, b1_ref, w1l_ref, tf_ref, w2_ref, b2_ref, x_ref,
                 o_ref, *, inv_b):
    j = pl.program_id(0)
    h = h_ref[...] + b1_ref[...] + tf_ref[...] * w1l_ref[...]
    hs = (h * jax.nn.sigmoid(h)).astype(jnp.bfloat16)
    pred = jax.lax.dot_general(
        hs, w2_ref[...], (((1,), (0,)), ((), ())),
        preferred_element_type=jnp.float32) + b2_ref[...]
    d = x_ref[...] - pred
    part = jnp.sum(d * d) * inv_b

    @pl.when(j == 0)
    def _():
        o_ref[...] = jnp.zeros_like(o_ref)

    o_ref[...] = o_ref[...] + part


def _mlp2_loss(h, b1_2d, w1_last, tf, w2, b2_2d, x):
    b, hid = h.shape
    n = w2.shape[1]
    bn = _blk(n, 1024)
    out = pl.pallas_call(
        functools.partial(_mlp2_kernel, inv_b=1.0 / b),
        grid=(n // bn,),
        in_specs=[
            pl.BlockSpec((b, hid), lambda j: (0, 0)),
            pl.BlockSpec((1, hid), lambda j: (0, 0)),
            pl.BlockSpec((1, hid), lambda j: (0, 0)),
            pl.BlockSpec((b, 1), lambda j: (0, 0)),
            pl.BlockSpec((hid, bn), lambda j: (0, j)),
            pl.BlockSpec((1, bn), lambda j: (0, j)),
            pl.BlockSpec((b, bn), lambda j: (0, j)),
        ],
        out_specs=pl.BlockSpec((1, 1), lambda j: (0, 0)),
        out_shape=jax.ShapeDtypeStruct((1, 1), jnp.float32),
        compiler_params=pltpu.CompilerParams(
            dimension_semantics=("arbitrary",)),
    )(h, b1_2d, w1_last, tf, w2, b2_2d, x)
    return out[0, 0]


def kernel(x, entity_emb, weight_velocity, W_second, b_second, W1, b1, W2, b2,
           adj_vals, eigen_val0, adj_rows, adj_cols, kg_src, kg_dst,
           batch_item_ids):
    b, n_items = x.shape
    n_ent, latdim = entity_emb.shape
    hid = W1.shape[1]
    np_ = _round_up(n_items, 128 if n_items < 1024 else 1024)
    pad = np_ - n_items

    key = jax.random.key(42)
    kt, kd = jax.random.split(key)
    t = jax.random.randint(kt, (b, 1), 1, _T + 1)
    tf = t.astype(jnp.float32)
    keep = jax.random.bernoulli(kd, 1.0 - _DROPOUT, x.shape)

    inv_keep = 1.0 / (1.0 - _DROPOUT)
    c = jnp.where(keep, inv_keep * x, 0.0)
    xp = jnp.pad(x, ((0, 0), (0, pad)))
    mbp = jnp.pad(jnp.concatenate([x, c], axis=0).astype(jnp.bfloat16),
                  ((0, 0), (0, pad)))

    a_dense = jnp.zeros((np_, np_), jnp.float32)
    a_dense = a_dense.at[adj_rows, adj_cols].add(adj_vals)
    a_dense = a_dense.astype(jnp.bfloat16)

    p = _matmul(mbp, a_dense, trans_b=True, out_dtype=jnp.bfloat16)
    s = _matmul(p, a_dense, trans_b=False, out_dtype=jnp.float32)

    esrc = entity_emb[kg_src].astype(jnp.bfloat16)
    edst = entity_emb[kg_dst].astype(jnp.bfloat16)
    msg = _kg_messages(esrc, edst, weight_velocity.astype(jnp.bfloat16))
    sec = jax.ops.segment_sum(msg, kg_src, num_segments=n_ent)
    secg = sec[batch_item_ids]

    gamma = _ALPHA * tf / _T
    inv_eig = (1.0 / eigen_val0).reshape(1, 1)
    wsp = jnp.pad(W_second.astype(jnp.bfloat16), ((0, 0), (0, pad)))
    bsp = jnp.pad(b_second.reshape(1, n_items), ((0, 0), (0, pad)))
    zs = _assemble(xp, mbp, s, secg, wsp, bsp, 1.0 - gamma,
                   gamma * inv_eig[0, 0], inv_eig)

    w1p = jnp.pad(W1[:3 * n_items].reshape(3, n_items, hid).astype(jnp.bfloat16),
                  ((0, 0), (0, pad), (0, 0)))
    h = _mlp1(zs, w1p)

    w2p = jnp.pad(W2.astype(jnp.bfloat16), ((0, 0), (0, pad)))
    b2p = jnp.pad(b2.reshape(1, n_items), ((0, 0), (0, pad)))
    loss = _mlp2_loss(h, b1.reshape(1, -1), W1[-1:, :], tf, w2p, b2p, xp)
    return loss


# speedup vs baseline: 5.2027x; 1.0140x over previous
import functools

import jax
import jax.numpy as jnp
from jax.experimental import pallas as pl
from jax.experimental.pallas import tpu as pltpu

_ALPHA = 2.5
_T = 2
_DROPOUT = 0.5


def _round_up(n, m):
    return ((n + m - 1) // m) * m


def _blk(n, target):
    b = min(n, target)
    while n % b:
        b -= 128
    return b


def _mm_kernel(a_ref, b_ref, o_ref, acc_ref, *, nk, trans_b):
    k = pl.program_id(2)

    @pl.when(k == 0)
    def _():
        acc_ref[...] = jnp.zeros_like(acc_ref)

    dn = (((1,), (1,)), ((), ())) if trans_b else (((1,), (0,)), ((), ()))
    acc_ref[...] += jax.lax.dot_general(
        a_ref[...], b_ref[...].astype(jnp.bfloat16), dn,
        preferred_element_type=jnp.float32)

    @pl.when(k == nk - 1)
    def _():
        o_ref[...] = acc_ref[...].astype(o_ref.dtype)


def _matmul(a, b, *, trans_b=False, out_dtype=jnp.float32, bm=None, bn=None,
            bk=None):
    m, ka = a.shape
    if trans_b:
        n, kb = b.shape
    else:
        kb, n = b.shape
    assert ka == kb
    bm = bm or m
    bn = bn or _blk(n, 1024)
    bk = bk or _blk(ka, 1024)
    nk = ka // bk
    b_spec = (pl.BlockSpec((bn, bk), lambda i, j, k: (j, k)) if trans_b
              else pl.BlockSpec((bk, bn), lambda i, j, k: (k, j)))
    return pl.pallas_call(
        functools.partial(_mm_kernel, nk=nk, trans_b=trans_b),
        grid=(m // bm, n // bn, nk),
        in_specs=[pl.BlockSpec((bm, bk), lambda i, j, k: (i, k)), b_spec],
        out_specs=pl.BlockSpec((bm, bn), lambda i, j, k: (i, j)),
        out_shape=jax.ShapeDtypeStruct((m, n), out_dtype),
        scratch_shapes=[pltpu.VMEM((bm, bn), jnp.float32)],
        compiler_params=pltpu.CompilerParams(
            dimension_semantics=("parallel", "parallel", "arbitrary")),
    )(a, b)


def _kg_msg_kernel(src_ref, dst_ref, wv_ref, o_ref):
    d = src_ref[...]
    dst = dst_ref[...]
    v = jax.lax.dot_general(
        (dst - d), wv_ref[...], (((1,), (0,)), ((), ())),
        preferred_element_type=jnp.float32)
    o_ref[...] = jnp.tanh(v) * dst.astype(jnp.float32)


def _kg_messages(esrc, edst, wv, be=2000):
    e, d = esrc.shape
    be = min(be, e)
    assert e % be == 0
    return pl.pallas_call(
        _kg_msg_kernel,
        grid=(e // be,),
        in_specs=[
            pl.BlockSpec((be, d), lambda i: (i, 0)),
            pl.BlockSpec((be, d), lambda i: (i, 0)),
            pl.BlockSpec((d, d), lambda i: (0, 0)),
        ],
        out_specs=pl.BlockSpec((be, d), lambda i: (i, 0)),
        out_shape=jax.ShapeDtypeStruct((e, d), jnp.float32),
        compiler_params=pltpu.CompilerParams(
            dimension_semantics=("arbitrary",)),
    )(esrc, edst, wv)


def _assemble_kernel(x_ref, c_ref, ax_ref, ac_ref, secg_ref, ws_ref, bs_ref,
                     g1_ref, g2_ref, inv_eig_ref, o_ref):
    x = x_ref[...]
    sec = jax.lax.dot_general(
        secg_ref[...], ws_ref[...], (((1,), (0,)), ((), ())),
        preferred_element_type=jnp.float32) + bs_ref[...]
    z = x * g1_ref[...] + g2_ref[...] * ax_ref[...] + sec
    o_ref[0, :, :] = z.astype(o_ref.dtype)
    o_ref[1, :, :] = c_ref[...].astype(o_ref.dtype)
    o_ref[2, :, :] = (ac_ref[...] * inv_eig_ref[...]).astype(o_ref.dtype)


def _assemble(x, c_bf, s, secg, w_second, b_second2d, g1, g2, inv_eig):
    b, n = x.shape
    d = secg.shape[1]
    bn = _blk(n, 1024)
    return pl.pallas_call(
        _assemble_kernel,
        grid=(n // bn,),
        in_specs=[
            pl.BlockSpec((b, bn), lambda j: (0, j)),
            pl.BlockSpec((b, bn), lambda j: (1, j)),
            pl.BlockSpec((b, bn), lambda j: (0, j)),
            pl.BlockSpec((b, bn), lambda j: (1, j)),
            pl.BlockSpec((b, d), lambda j: (0, 0)),
            pl.BlockSpec((d, bn), lambda j: (0, j)),
            pl.BlockSpec((1, bn), lambda j: (0, j)),
            pl.BlockSpec((b, 1), lambda j: (0, 0)),
            pl.BlockSpec((b, 1), lambda j: (0, 0)),
            pl.BlockSpec((1, 1), lambda j: (0, 0)),
        ],
        out_specs=pl.BlockSpec((3, b, bn), lambda j: (0, 0, j)),
        out_shape=jax.ShapeDtypeStruct((3, b, n), jnp.bfloat16),
        compiler_params=pltpu.CompilerParams(
            dimension_semantics=("arbitrary",)),
    )(x, c_bf, s, s, secg, w_second, b_second2d, g1, g2, inv_eig)


def _w1_pack_kernel(w1_ref, o_ref, *, nk):
    k = pl.program_id(1)
    blk = w1_ref[...].astype(jnp.bfloat16)
    o_ref[0] = jnp.where(k < nk, blk, jnp.zeros_like(blk))


def _w1_pack(w1, n_items, np_, hid, bk=1000):
    bk = min(bk, n_items)
    nk = n_items // bk
    nk_pad = -(-np_ // bk)

    def idx_in(s, k):
        kk = jnp.minimum(k, nk - 1)
        return (s * nk + kk, 0)

    return pl.pallas_call(
        functools.partial(_w1_pack_kernel, nk=nk),
        grid=(3, nk_pad),
        in_specs=[pl.BlockSpec((bk, hid), idx_in)],
        out_specs=pl.BlockSpec((1, bk, hid), lambda s, k: (s, k, 0)),
        out_shape=jax.ShapeDtypeStruct((3, np_, hid), jnp.bfloat16),
        compiler_params=pltpu.CompilerParams(
            dimension_semantics=("arbitrary", "arbitrary")),
    )(w1)


def _mlp1_kernel(z_ref, w1_ref, o_ref, acc_ref, *, ns, nk):
    s = pl.program_id(0)
    k = pl.program_id(1)

    @pl.when((s == 0) & (k == 0))
    def _():
        acc_ref[...] = jnp.zeros_like(acc_ref)

    acc_ref[...] += jax.lax.dot_general(
        z_ref[0], w1_ref[0], (((1,), (0,)), ((), ())),
        preferred_element_type=jnp.float32)

    @pl.when((s == ns - 1) & (k == nk - 1))
    def _():
        o_ref[...] = acc_ref[...]


def _mlp1(zs, w1p):
    ns, b, n = zs.shape
    hid = w1p.shape[2]
    bk = _blk(n, 1024)
    nk = n // bk
    return pl.pallas_call(
        functools.partial(_mlp1_kernel, ns=ns, nk=nk),
        grid=(ns, nk),
        in_specs=[
            pl.BlockSpec((1, b, bk), lambda s, k: (s, 0, k)),
            pl.BlockSpec((1, bk, hid), lambda s, k: (s, k, 0)),
        ],
        out_specs=pl.BlockSpec((b, hid), lambda s, k: (0, 0)),
        out_shape=jax.ShapeDtypeStruct((b, hid), jnp.float32),
        scratch_shapes=[pltpu.VMEM((b, hid), jnp.float32)],
        compiler_params=pltpu.CompilerParams(
            dimension_semantics=("arbitrary", "arbitrary")),
    )(zs, w1p)


def _mlp2_kernel(h_ref, b1_ref, w1l_ref, tf_ref, w2_ref, b2_ref, x_ref,
                 o_ref, *, inv_b):
    j = pl.program_id(0)
    h = h_ref[...] + b1_ref[...] + tf_ref[...] * w1l_ref[...]
    hs = (h * jax.nn.sigmoid(h)).astype(jnp.bfloat16)
    pred = jax.lax.dot_general(
        hs, w2_ref[...], (((1,), (0,)), ((), ())),
        preferred_element_type=jnp.float32) + b2_ref[...]
    d = x_ref[...] - pred
    part = jnp.sum(d * d) * inv_b

    @pl.when(j == 0)
    def _():
        o_ref[...] = jnp.zeros_like(o_ref)

    o_ref[...] = o_ref[...] + part


def _mlp2_loss(h, b1_2d, w1_last, tf, w2, b2_2d, x):
    b, hid = h.shape
    n = w2.shape[1]
    bn = _blk(n, 1024)
    out = pl.pallas_call(
        functools.partial(_mlp2_kernel, inv_b=1.0 / b),
        grid=(n // bn,),
        in_specs=[
            pl.BlockSpec((b, hid), lambda j: (0, 0)),
            pl.BlockSpec((1, hid), lambda j: (0, 0)),
            pl.BlockSpec((1, hid), lambda j: (0, 0)),
            pl.BlockSpec((b, 1), lambda j: (0, 0)),
            pl.BlockSpec((hid, bn), lambda j: (0, j)),
            pl.BlockSpec((1, bn), lambda j: (0, j)),
            pl.BlockSpec((b, bn), lambda j: (0, j)),
        ],
        out_specs=pl.BlockSpec((1, 1), lambda j: (0, 0)),
        out_shape=jax.ShapeDtypeStruct((1, 1), jnp.float32),
        compiler_params=pltpu.CompilerParams(
            dimension_semantics=("arbitrary",)),
    )(h, b1_2d, w1_last, tf, w2, b2_2d, x)
    return out[0, 0]


def kernel(x, entity_emb, weight_velocity, W_second, b_second, W1, b1, W2, b2,
           adj_vals, eigen_val0, adj_rows, adj_cols, kg_src, kg_dst,
           batch_item_ids):
    b, n_items = x.shape
    n_ent, latdim = entity_emb.shape
    hid = W1.shape[1]
    np_ = _round_up(n_items, 128 if n_items < 1024 else 1024)
    pad = np_ - n_items

    key = jax.random.key(42)
    kt, kd = jax.random.split(key)
    t = jax.random.randint(kt, (b, 1), 1, _T + 1)
    tf = t.astype(jnp.float32)
    keep = jax.random.bernoulli(kd, 1.0 - _DROPOUT, x.shape)

    inv_keep = 1.0 / (1.0 - _DROPOUT)
    c = jnp.where(keep, inv_keep * x, 0.0)
    xp = jnp.pad(x, ((0, 0), (0, pad)))
    mbp = jnp.pad(jnp.concatenate([x, c], axis=0).astype(jnp.bfloat16),
                  ((0, 0), (0, pad)))

    a_dense = jnp.zeros((np_, np_), jnp.float32)
    a_dense = a_dense.at[adj_rows, adj_cols].add(adj_vals)

    p = _matmul(mbp, a_dense, trans_b=True, out_dtype=jnp.bfloat16)
    s = _matmul(p, a_dense, trans_b=False, out_dtype=jnp.float32)

    esrc = entity_emb[kg_src].astype(jnp.bfloat16)
    edst = entity_emb[kg_dst].astype(jnp.bfloat16)
    msg = _kg_messages(esrc, edst, weight_velocity.astype(jnp.bfloat16))
    sec = jax.ops.segment_sum(msg, kg_src, num_segments=n_ent)
    secg = sec[batch_item_ids]

    gamma = _ALPHA * tf / _T
    inv_eig = (1.0 / eigen_val0).reshape(1, 1)
    wsp = jnp.pad(W_second.astype(jnp.bfloat16), ((0, 0), (0, pad)))
    bsp = jnp.pad(b_second.reshape(1, n_items), ((0, 0), (0, pad)))
    zs = _assemble(xp, mbp, s, secg, wsp, bsp, 1.0 - gamma,
                   gamma * inv_eig[0, 0], inv_eig)

    w1p = _w1_pack(W1, n_items, np_, hid)
    h = _mlp1(zs, w1p)

    w2p = jnp.pad(W2.astype(jnp.bfloat16), ((0, 0), (0, pad)))
    b2p = jnp.pad(b2.reshape(1, n_items), ((0, 0), (0, pad)))
    loss = _mlp2_loss(h, b1.reshape(1, -1), W1[-1:, :], tf, w2p, b2p, xp)
    return loss


# speedup vs baseline: 6.3958x; 1.2293x over previous
import functools

import jax
import jax.numpy as jnp
from jax import lax
from jax.experimental import pallas as pl
from jax.experimental.pallas import tpu as pltpu
from jax.experimental.pallas import tpu_sc as plsc

_ALPHA = 2.5
_T = 2
_DROPOUT = 0.5


def _round_up(n, m):
    return ((n + m - 1) // m) * m


def _blk(n, target):
    b = min(n, target)
    while n % b:
        b -= 128
    return b


def _mm_kernel(a_ref, b_ref, o_ref, acc_ref, *, nk, trans_b):
    k = pl.program_id(2)

    @pl.when(k == 0)
    def _():
        acc_ref[...] = jnp.zeros_like(acc_ref)

    dn = (((1,), (1,)), ((), ())) if trans_b else (((1,), (0,)), ((), ()))
    acc_ref[...] += jax.lax.dot_general(
        a_ref[...], b_ref[...].astype(jnp.bfloat16), dn,
        preferred_element_type=jnp.float32)

    @pl.when(k == nk - 1)
    def _():
        o_ref[...] = acc_ref[...].astype(o_ref.dtype)


def _matmul(a, b, *, trans_b=False, out_dtype=jnp.float32, bm=None, bn=None,
            bk=None):
    m, ka = a.shape
    if trans_b:
        n, kb = b.shape
    else:
        kb, n = b.shape
    assert ka == kb
    bm = bm or m
    bn = bn or _blk(n, 1024)
    bk = bk or _blk(ka, 1024)
    nk = ka // bk
    b_spec = (pl.BlockSpec((bn, bk), lambda i, j, k: (j, k)) if trans_b
              else pl.BlockSpec((bk, bn), lambda i, j, k: (k, j)))
    return pl.pallas_call(
        functools.partial(_mm_kernel, nk=nk, trans_b=trans_b),
        grid=(m // bm, n // bn, nk),
        in_specs=[pl.BlockSpec((bm, bk), lambda i, j, k: (i, k)), b_spec],
        out_specs=pl.BlockSpec((bm, bn), lambda i, j, k: (i, j)),
        out_shape=jax.ShapeDtypeStruct((m, n), out_dtype),
        scratch_shapes=[pltpu.VMEM((bm, bn), jnp.float32)],
        compiler_params=pltpu.CompilerParams(
            dimension_semantics=("parallel", "parallel", "arbitrary")),
    )(a, b)


_NC, _NS = 2, 16
_EPW = 5120
_ACC_ROWS = _NS * 72


def _kg_sc_body(d1_hbm, emb_hbm, src_hbm, dst_hbm, inv_hbm, out_hbm,
                inv_v, srcb, dstb, ks1d, kd1d, km1d, m2d,
                g_d1d, g_d1s, g_emb, msg_v, acc_sh, sem1, sem2, sem3):
    c = lax.axis_index("c")
    s = lax.axis_index("s")
    wid = c * _NS + s

    def zrow(r, _):
        for l in range(8):
            msg_v[r, pl.ds(l * 16, 16)] = jnp.zeros((16,), jnp.float32)
        return 0

    lax.fori_loop(0, 128, zrow, 0)
    pltpu.sync_copy(msg_v.at[pl.ds(0, 72)], acc_sh.at[pl.ds(s * 72, 72)])
    plsc.subcore_barrier()

    pltpu.sync_copy(inv_hbm, inv_v)
    pltpu.sync_copy(src_hbm.at[pl.ds(wid * 40, 40)], srcb)
    pltpu.sync_copy(dst_hbm.at[pl.ds(wid * 40, 40)], dstb)

    def pre(i, _):
        z = jnp.zeros((16,), jnp.int32)
        ks1d[pl.ds(i * 16, 16)] = z
        kd1d[pl.ds(i * 16, 16)] = z
        km1d[pl.ds(i * 16, 16)] = z + (_ACC_ROWS - 16)
        return 0

    lax.fori_loop(0, _EPW // 16, pre, 0)

    def comp(i, off):
        r = i // 8
        l = i % 8
        sv = srcb[r, pl.ds(l * 16, 16)]
        dv = dstb[r, pl.ds(l * 16, 16)]
        mv = plsc.load_gather(inv_v, [sv])
        mask = mv >= 0
        plsc.store_compressed(ks1d.at[pl.ds(off, 16)], sv, mask=mask)
        plsc.store_compressed(kd1d.at[pl.ds(off, 16)], dv, mask=mask)
        plsc.store_compressed(km1d.at[pl.ds(off, 16)], mv, mask=mask)
        return off + jnp.sum(mask.astype(jnp.int32))

    kcount = lax.fori_loop(0, _EPW // 16, comp, jnp.int32(0))

    def relay(i, _):
        r = i // 8
        l = i % 8
        m2d[r, pl.ds(l * 16, 16)] = km1d[pl.ds(i * 16, 16)]
        return 0

    lax.fori_loop(0, _EPW // 16, relay, 0)

    def chunk(ch, _):
        i0 = ch * 128
        cp1 = pltpu.async_copy(d1_hbm.at[kd1d.at[pl.ds(i0, 128)]], g_d1d, sem1)
        cp2 = pltpu.async_copy(d1_hbm.at[ks1d.at[pl.ds(i0, 128)]], g_d1s, sem2)
        cp3 = pltpu.async_copy(emb_hbm.at[kd1d.at[pl.ds(i0, 128)]], g_emb, sem3)
        cp1.wait()
        cp2.wait()
        cp3.wait()

        def mrow(r, _):
            for l in range(8):
                sl = pl.ds(l * 16, 16)
                x2 = 2.0 * (g_d1d[r, sl] - g_d1s[r, sl])
                th = 1.0 - 2.0 / (jnp.exp(x2) + 1.0)
                msg_v[r, sl] = th * g_emb[r, sl]
            return 0

        lax.fori_loop(0, 128, mrow, 0)
        pltpu.sync_copy(msg_v, acc_sh.at[m2d.at[ch]], add=True)
        return 0

    nch = (kcount + 127) // 128
    lax.fori_loop(0, nch, chunk, 0)

    plsc.subcore_barrier()
    pltpu.sync_copy(acc_sh.at[pl.ds(s * 72, 72)],
                    out_hbm.at[c, pl.ds(s * 72, 72)])


def _kg_divergence_sc(d1, emb, src_p, dst_p, inv):
    mesh = plsc.VectorSubcoreMesh(core_axis_name="c", subcore_axis_name="s",
                                  num_cores=_NC, num_subcores=_NS)
    latdim = emb.shape[1]
    f32 = jnp.float32
    return pl.kernel(
        _kg_sc_body,
        out_type=jax.ShapeDtypeStruct((_NC, _ACC_ROWS, latdim), f32),
        mesh=mesh,
        scratch_types=[
            pltpu.VMEM(inv.shape, jnp.int32),
            pltpu.VMEM((40, 128), jnp.int32),
            pltpu.VMEM((40, 128), jnp.int32),
            pltpu.VMEM((_EPW,), jnp.int32),
            pltpu.VMEM((_EPW,), jnp.int32),
            pltpu.VMEM((_EPW,), jnp.int32),
            pltpu.VMEM((40, 128), jnp.int32),
            pltpu.VMEM((128, latdim), f32),
            pltpu.VMEM((128, latdim), f32),
            pltpu.VMEM((128, latdim), f32),
            pltpu.VMEM((128, latdim), f32),
            pltpu.VMEM_SHARED((_ACC_ROWS, latdim), f32),
            pltpu.SemaphoreType.DMA,
            pltpu.SemaphoreType.DMA,
            pltpu.SemaphoreType.DMA,
        ],
        compiler_params=pltpu.CompilerParams(needs_layout_passes=False),
    )(d1, emb, src_p, dst_p, inv)


def _assemble_kernel(x_ref, c_ref, ax_ref, ac_ref, secg_ref, ws_ref, bs_ref,
                     g1_ref, g2_ref, inv_eig_ref, o_ref):
    x = x_ref[...]
    sec = jax.lax.dot_general(
        secg_ref[...], ws_ref[...], (((1,), (0,)), ((), ())),
        preferred_element_type=jnp.float32) + bs_ref[...]
    z = x * g1_ref[...] + g2_ref[...] * ax_ref[...] + sec
    o_ref[0, :, :] = z.astype(o_ref.dtype)
    o_ref[1, :, :] = c_ref[...].astype(o_ref.dtype)
    o_ref[2, :, :] = (ac_ref[...] * inv_eig_ref[...]).astype(o_ref.dtype)


def _assemble(x, c_bf, s, secg, w_second, b_second2d, g1, g2, inv_eig):
    b, n = x.shape
    d = secg.shape[1]
    bn = _blk(n, 1024)
    return pl.pallas_call(
        _assemble_kernel,
        grid=(n // bn,),
        in_specs=[
            pl.BlockSpec((b, bn), lambda j: (0, j)),
            pl.BlockSpec((b, bn), lambda j: (1, j)),
            pl.BlockSpec((b, bn), lambda j: (0, j)),
            pl.BlockSpec((b, bn), lambda j: (1, j)),
            pl.BlockSpec((b, d), lambda j: (0, 0)),
            pl.BlockSpec((d, bn), lambda j: (0, j)),
            pl.BlockSpec((1, bn), lambda j: (0, j)),
            pl.BlockSpec((b, 1), lambda j: (0, 0)),
            pl.BlockSpec((b, 1), lambda j: (0, 0)),
            pl.BlockSpec((1, 1), lambda j: (0, 0)),
        ],
        out_specs=pl.BlockSpec((3, b, bn), lambda j: (0, 0, j)),
        out_shape=jax.ShapeDtypeStruct((3, b, n), jnp.bfloat16),
        compiler_params=pltpu.CompilerParams(
            dimension_semantics=("arbitrary",)),
    )(x, c_bf, s, s, secg, w_second, b_second2d, g1, g2, inv_eig)


def _w1_pack_kernel(w1_ref, o_ref, *, nk):
    k = pl.program_id(1)
    blk = w1_ref[...].astype(jnp.bfloat16)
    o_ref[0] = jnp.where(k < nk, blk, jnp.zeros_like(blk))


def _w1_pack(w1, n_items, np_, hid, bk=1000):
    bk = min(bk, n_items)
    nk = n_items // bk
    nk_pad = -(-np_ // bk)

    def idx_in(s, k):
        kk = jnp.minimum(k, nk - 1)
        return (s * nk + kk, 0)

    return pl.pallas_call(
        functools.partial(_w1_pack_kernel, nk=nk),
        grid=(3, nk_pad),
        in_specs=[pl.BlockSpec((bk, hid), idx_in)],
        out_specs=pl.BlockSpec((1, bk, hid), lambda s, k: (s, k, 0)),
        out_shape=jax.ShapeDtypeStruct((3, np_, hid), jnp.bfloat16),
        compiler_params=pltpu.CompilerParams(
            dimension_semantics=("arbitrary", "arbitrary")),
    )(w1)


def _mlp1_kernel(z_ref, w1_ref, o_ref, acc_ref, *, ns, nk):
    s = pl.program_id(0)
    k = pl.program_id(1)

    @pl.when((s == 0) & (k == 0))
    def _():
        acc_ref[...] = jnp.zeros_like(acc_ref)

    acc_ref[...] += jax.lax.dot_general(
        z_ref[0], w1_ref[0], (((1,), (0,)), ((), ())),
        preferred_element_type=jnp.float32)

    @pl.when((s == ns - 1) & (k == nk - 1))
    def _():
        o_ref[...] = acc_ref[...]


def _mlp1(zs, w1p):
    ns, b, n = zs.shape
    hid = w1p.shape[2]
    bk = _blk(n, 1024)
    nk = n // bk
    return pl.pallas_call(
        functools.partial(_mlp1_kernel, ns=ns, nk=nk),
        grid=(ns, nk),
        in_specs=[
            pl.BlockSpec((1, b, bk), lambda s, k: (s, 0, k)),
            pl.BlockSpec((1, bk, hid), lambda s, k: (s, k, 0)),
        ],
        out_specs=pl.BlockSpec((b, hid), lambda s, k: (0, 0)),
        out_shape=jax.ShapeDtypeStruct((b, hid), jnp.float32),
        scratch_shapes=[pltpu.VMEM((b, hid), jnp.float32)],
        compiler_params=pltpu.CompilerParams(
            dimension_semantics=("arbitrary", "arbitrary")),
    )(zs, w1p)


def _mlp2_kernel(h_ref, b1_ref, w1l_ref, tf_ref, w2_ref, b2_ref, x_ref,
                 o_ref, *, inv_b):
    j = pl.program_id(0)
    h = h_ref[...] + b1_ref[...] + tf_ref[...] * w1l_ref[...]
    hs = (h * jax.nn.sigmoid(h)).astype(jnp.bfloat16)
    pred = jax.lax.dot_general(
        hs, w2_ref[...], (((1,), (0,)), ((), ())),
        preferred_element_type=jnp.float32) + b2_ref[...]
    d = x_ref[...] - pred
    part = jnp.sum(d * d) * inv_b

    @pl.when(j == 0)
    def _():
        o_ref[...] = jnp.zeros_like(o_ref)

    o_ref[...] = o_ref[...] + part


def _mlp2_loss(h, b1_2d, w1_last, tf, w2, b2_2d, x):
    b, hid = h.shape
    n = w2.shape[1]
    bn = _blk(n, 1024)
    out = pl.pallas_call(
        functools.partial(_mlp2_kernel, inv_b=1.0 / b),
        grid=(n // bn,),
        in_specs=[
            pl.BlockSpec((b, hid), lambda j: (0, 0)),
            pl.BlockSpec((1, hid), lambda j: (0, 0)),
            pl.BlockSpec((1, hid), lambda j: (0, 0)),
            pl.BlockSpec((b, 1), lambda j: (0, 0)),
            pl.BlockSpec((hid, bn), lambda j: (0, j)),
            pl.BlockSpec((1, bn), lambda j: (0, j)),
            pl.BlockSpec((b, bn), lambda j: (0, j)),
        ],
        out_specs=pl.BlockSpec((1, 1), lambda j: (0, 0)),
        out_shape=jax.ShapeDtypeStruct((1, 1), jnp.float32),
        compiler_params=pltpu.CompilerParams(
            dimension_semantics=("arbitrary",)),
    )(h, b1_2d, w1_last, tf, w2, b2_2d, x)
    return out[0, 0]


def kernel(x, entity_emb, weight_velocity, W_second, b_second, W1, b1, W2, b2,
           adj_vals, eigen_val0, adj_rows, adj_cols, kg_src, kg_dst,
           batch_item_ids):
    b, n_items = x.shape
    n_ent, latdim = entity_emb.shape
    hid = W1.shape[1]
    np_ = _round_up(n_items, 128 if n_items < 1024 else 1024)
    pad = np_ - n_items

    key = jax.random.key(42)
    kt, kd = jax.random.split(key)
    t = jax.random.randint(kt, (b, 1), 1, _T + 1)
    tf = t.astype(jnp.float32)
    keep = jax.random.bernoulli(kd, 1.0 - _DROPOUT, x.shape)

    inv_keep = 1.0 / (1.0 - _DROPOUT)
    c = jnp.where(keep, inv_keep * x, 0.0)
    xp = jnp.pad(x, ((0, 0), (0, pad)))
    mbp = jnp.pad(jnp.concatenate([x, c], axis=0).astype(jnp.bfloat16),
                  ((0, 0), (0, pad)))

    a_dense = jnp.zeros((np_, np_), jnp.float32)
    a_dense = a_dense.at[adj_rows, adj_cols].add(adj_vals)

    p = _matmul(mbp, a_dense, trans_b=True, out_dtype=jnp.bfloat16)
    s = _matmul(p, a_dense, trans_b=False, out_dtype=jnp.float32)

    d1 = _matmul(entity_emb.astype(jnp.bfloat16), weight_velocity)
    e = kg_src.shape[0]
    epad = _NC * _NS * _EPW
    src_p = jnp.concatenate(
        [kg_src.astype(jnp.int32),
         jnp.full((epad - e,), n_ent, jnp.int32)]).reshape(epad // 128, 128)
    dst_p = jnp.concatenate(
        [kg_dst.astype(jnp.int32),
         jnp.zeros((epad - e,), jnp.int32)]).reshape(epad // 128, 128)
    inv = jnp.full((_round_up(n_ent + 1, 8),), -1, jnp.int32)
    inv = inv.at[batch_item_ids].set(jnp.arange(b, dtype=jnp.int32))
    out_sc = _kg_divergence_sc(d1, entity_emb, src_p, dst_p, inv)
    acc_tot = out_sc[0, :b] + out_sc[1, :b]
    secg = acc_tot[inv[batch_item_ids]]

    gamma = _ALPHA * tf / _T
    inv_eig = (1.0 / eigen_val0).reshape(1, 1)
    wsp = jnp.pad(W_second.astype(jnp.bfloat16), ((0, 0), (0, pad)))
    bsp = jnp.pad(b_second.reshape(1, n_items), ((0, 0), (0, pad)))
    zs = _assemble(xp, mbp, s, secg, wsp, bsp, 1.0 - gamma,
                   gamma * inv_eig[0, 0], inv_eig)

    w1p = _w1_pack(W1, n_items, np_, hid)
    h = _mlp1(zs, w1p)

    w2p = jnp.pad(W2.astype(jnp.bfloat16), ((0, 0), (0, pad)))
    b2p = jnp.pad(b2.reshape(1, n_items), ((0, 0), (0, pad)))
    loss = _mlp2_loss(h, b1.reshape(1, -1), W1[-1:, :], tf, w2p, b2p, xp)
    return loss


# speedup vs baseline: 6.4409x; 1.0070x over previous
import functools

import jax
import jax.numpy as jnp
from jax import lax
from jax.experimental import pallas as pl
from jax.experimental.pallas import tpu as pltpu
from jax.experimental.pallas import tpu_sc as plsc

_ALPHA = 2.5
_T = 2
_DROPOUT = 0.5


def _round_up(n, m):
    return ((n + m - 1) // m) * m


def _blk(n, target):
    b = min(n, target)
    while n % b:
        b -= 128
    return b


def _mm_kernel(a_ref, b_ref, o_ref, acc_ref, *, nk, trans_b):
    k = pl.program_id(2)

    @pl.when(k == 0)
    def _():
        acc_ref[...] = jnp.zeros_like(acc_ref)

    dn = (((1,), (1,)), ((), ())) if trans_b else (((1,), (0,)), ((), ()))
    acc_ref[...] += jax.lax.dot_general(
        a_ref[...], b_ref[...].astype(jnp.bfloat16), dn,
        preferred_element_type=jnp.float32)

    @pl.when(k == nk - 1)
    def _():
        o_ref[...] = acc_ref[...].astype(o_ref.dtype)


def _matmul(a, b, *, trans_b=False, out_dtype=jnp.float32, bm=None, bn=None,
            bk=None):
    m, ka = a.shape
    if trans_b:
        n, kb = b.shape
    else:
        kb, n = b.shape
    assert ka == kb
    bm = bm or m
    bn = bn or _blk(n, 1024)
    bk = bk or _blk(ka, 1024)
    nk = ka // bk
    b_spec = (pl.BlockSpec((bn, bk), lambda i, j, k: (j, k)) if trans_b
              else pl.BlockSpec((bk, bn), lambda i, j, k: (k, j)))
    return pl.pallas_call(
        functools.partial(_mm_kernel, nk=nk, trans_b=trans_b),
        grid=(m // bm, n // bn, nk),
        in_specs=[pl.BlockSpec((bm, bk), lambda i, j, k: (i, k)), b_spec],
        out_specs=pl.BlockSpec((bm, bn), lambda i, j, k: (i, j)),
        out_shape=jax.ShapeDtypeStruct((m, n), out_dtype),
        scratch_shapes=[pltpu.VMEM((bm, bn), jnp.float32)],
        compiler_params=pltpu.CompilerParams(
            dimension_semantics=("parallel", "parallel", "arbitrary")),
    )(a, b)


_NC, _NS = 2, 16
_EPW = 5120
_ACC_ROWS = _NS * 72


def _kg_sc_body(d1_hbm, emb_hbm, src_hbm, dst_hbm, inv_hbm, out_hbm,
                inv_v, srcb, dstb, ks1d, kd1d, km1d, m2d,
                g_d1d, g_d1s, g_emb, msg_v, acc_sh, sem1, sem2, sem3,
                *, n_ent, e_per_w):
    c = lax.axis_index("c")
    s = lax.axis_index("s")
    wid = c * _NS + s

    def zrow(r, _):
        for l in range(8):
            msg_v[r, pl.ds(l * 16, 16)] = jnp.zeros((16,), jnp.float32)
        return 0

    lax.fori_loop(0, 128, zrow, 0)
    pltpu.sync_copy(msg_v.at[pl.ds(0, 72)], acc_sh.at[pl.ds(s * 72, 72)])
    plsc.subcore_barrier()

    def pre(i, _):
        z = jnp.zeros((16,), jnp.int32)
        srcb[pl.ds(i * 16, 16)] = z + n_ent
        dstb[pl.ds(i * 16, 16)] = z
        ks1d[pl.ds(i * 16, 16)] = z
        kd1d[pl.ds(i * 16, 16)] = z
        km1d[pl.ds(i * 16, 16)] = z + (_ACC_ROWS - 16)
        return 0

    lax.fori_loop(0, _EPW // 16, pre, 0)

    pltpu.sync_copy(inv_hbm, inv_v)
    pltpu.sync_copy(src_hbm.at[pl.ds(wid * e_per_w, e_per_w)],
                    srcb.at[pl.ds(0, e_per_w)])
    pltpu.sync_copy(dst_hbm.at[pl.ds(wid * e_per_w, e_per_w)],
                    dstb.at[pl.ds(0, e_per_w)])

    def comp(i, off):
        sv = srcb[pl.ds(i * 16, 16)]
        dv = dstb[pl.ds(i * 16, 16)]
        mv = plsc.load_gather(inv_v, [sv])
        mask = mv >= 0
        plsc.store_compressed(ks1d.at[pl.ds(off, 16)], sv, mask=mask)
        plsc.store_compressed(kd1d.at[pl.ds(off, 16)], dv, mask=mask)
        plsc.store_compressed(km1d.at[pl.ds(off, 16)], mv, mask=mask)
        return off + jnp.sum(mask.astype(jnp.int32))

    kcount = lax.fori_loop(0, _EPW // 16, comp, jnp.int32(0))

    def relay(i, _):
        r = i // 8
        ll = (i % 8) * 16
        m2d[r, pl.ds(ll, 16)] = km1d[pl.ds(i * 16, 16)]
        return 0

    lax.fori_loop(0, _EPW // 16, relay, 0)

    def chunk(ch, _):
        i0 = ch * 128
        cp1 = pltpu.async_copy(d1_hbm.at[kd1d.at[pl.ds(i0, 128)]], g_d1d, sem1)
        cp2 = pltpu.async_copy(d1_hbm.at[ks1d.at[pl.ds(i0, 128)]], g_d1s, sem2)
        cp3 = pltpu.async_copy(emb_hbm.at[kd1d.at[pl.ds(i0, 128)]], g_emb, sem3)
        cp1.wait()
        cp2.wait()
        cp3.wait()

        def mrow(r, _):
            for l in range(8):
                sl = pl.ds(l * 16, 16)
                x2 = 2.0 * (g_d1d[r, sl] - g_d1s[r, sl])
                th = 1.0 - 2.0 / (jnp.exp(x2) + 1.0)
                msg_v[r, sl] = th * g_emb[r, sl]
            return 0

        lax.fori_loop(0, 128, mrow, 0)
        pltpu.sync_copy(msg_v, acc_sh.at[m2d.at[ch]], add=True)
        return 0

    nch = (kcount + 127) // 128
    lax.fori_loop(0, nch, chunk, 0)

    plsc.subcore_barrier()
    pltpu.sync_copy(acc_sh.at[pl.ds(s * 72, 72)],
                    out_hbm.at[c, pl.ds(s * 72, 72)])


def _kg_divergence_sc(d1, emb, src_p, dst_p, inv):
    mesh = plsc.VectorSubcoreMesh(core_axis_name="c", subcore_axis_name="s",
                                  num_cores=_NC, num_subcores=_NS)
    latdim = emb.shape[1]
    f32 = jnp.float32
    e = src_p.shape[0]
    e_per_w = e // (_NC * _NS)
    assert e_per_w % 8 == 0 and e_per_w <= _EPW
    return pl.kernel(
        functools.partial(_kg_sc_body, n_ent=emb.shape[0], e_per_w=e_per_w),
        out_type=jax.ShapeDtypeStruct((_NC, _ACC_ROWS, latdim), f32),
        mesh=mesh,
        scratch_types=[
            pltpu.VMEM(inv.shape, jnp.int32),
            pltpu.VMEM((_EPW,), jnp.int32),
            pltpu.VMEM((_EPW,), jnp.int32),
            pltpu.VMEM((_EPW,), jnp.int32),
            pltpu.VMEM((_EPW,), jnp.int32),
            pltpu.VMEM((_EPW,), jnp.int32),
            pltpu.VMEM((40, 128), jnp.int32),
            pltpu.VMEM((128, latdim), f32),
            pltpu.VMEM((128, latdim), f32),
            pltpu.VMEM((128, latdim), f32),
            pltpu.VMEM((128, latdim), f32),
            pltpu.VMEM_SHARED((_ACC_ROWS, latdim), f32),
            pltpu.SemaphoreType.DMA,
            pltpu.SemaphoreType.DMA,
            pltpu.SemaphoreType.DMA,
        ],
        compiler_params=pltpu.CompilerParams(needs_layout_passes=False),
    )(d1, emb, src_p, dst_p, inv)


def _assemble_kernel(x_ref, c_ref, ax_ref, ac_ref, secg_ref, ws_ref, bs_ref,
                     g1_ref, g2_ref, inv_eig_ref, o_ref):
    x = x_ref[...]
    sec = jax.lax.dot_general(
        secg_ref[...], ws_ref[...], (((1,), (0,)), ((), ())),
        preferred_element_type=jnp.float32) + bs_ref[...]
    z = x * g1_ref[...] + g2_ref[...] * ax_ref[...] + sec
    o_ref[0, :, :] = z.astype(o_ref.dtype)
    o_ref[1, :, :] = c_ref[...].astype(o_ref.dtype)
    o_ref[2, :, :] = (ac_ref[...] * inv_eig_ref[...]).astype(o_ref.dtype)


def _assemble(x, c_bf, s, secg, w_second, b_second2d, g1, g2, inv_eig):
    b, n = x.shape
    d = secg.shape[1]
    bn = _blk(n, 1024)
    return pl.pallas_call(
        _assemble_kernel,
        grid=(n // bn,),
        in_specs=[
            pl.BlockSpec((b, bn), lambda j: (0, j)),
            pl.BlockSpec((b, bn), lambda j: (1, j)),
            pl.BlockSpec((b, bn), lambda j: (0, j)),
            pl.BlockSpec((b, bn), lambda j: (1, j)),
            pl.BlockSpec((b, d), lambda j: (0, 0)),
            pl.BlockSpec((d, bn), lambda j: (0, j)),
            pl.BlockSpec((1, bn), lambda j: (0, j)),
            pl.BlockSpec((b, 1), lambda j: (0, 0)),
            pl.BlockSpec((b, 1), lambda j: (0, 0)),
            pl.BlockSpec((1, 1), lambda j: (0, 0)),
        ],
        out_specs=pl.BlockSpec((3, b, bn), lambda j: (0, 0, j)),
        out_shape=jax.ShapeDtypeStruct((3, b, n), jnp.bfloat16),
        compiler_params=pltpu.CompilerParams(
            dimension_semantics=("arbitrary",)),
    )(x, c_bf, s, s, secg, w_second, b_second2d, g1, g2, inv_eig)


def _w1_pack_kernel(w1_ref, o_ref, *, nk):
    k = pl.program_id(1)
    blk = w1_ref[...].astype(jnp.bfloat16)
    o_ref[0] = jnp.where(k < nk, blk, jnp.zeros_like(blk))


def _w1_pack(w1, n_items, np_, hid, bk=1000):
    bk = min(bk, n_items)
    nk = n_items // bk
    nk_pad = -(-np_ // bk)

    def idx_in(s, k):
        kk = jnp.minimum(k, nk - 1)
        return (s * nk + kk, 0)

    return pl.pallas_call(
        functools.partial(_w1_pack_kernel, nk=nk),
        grid=(3, nk_pad),
        in_specs=[pl.BlockSpec((bk, hid), idx_in)],
        out_specs=pl.BlockSpec((1, bk, hid), lambda s, k: (s, k, 0)),
        out_shape=jax.ShapeDtypeStruct((3, np_, hid), jnp.bfloat16),
        compiler_params=pltpu.CompilerParams(
            dimension_semantics=("arbitrary", "arbitrary")),
    )(w1)


def _mlp1_kernel(z_ref, w1_ref, o_ref, acc_ref, *, ns, nk):
    s = pl.program_id(0)
    k = pl.program_id(1)

    @pl.when((s == 0) & (k == 0))
    def _():
        acc_ref[...] = jnp.zeros_like(acc_ref)

    acc_ref[...] += jax.lax.dot_general(
        z_ref[0], w1_ref[0], (((1,), (0,)), ((), ())),
        preferred_element_type=jnp.float32)

    @pl.when((s == ns - 1) & (k == nk - 1))
    def _():
        o_ref[...] = acc_ref[...]


def _mlp1(zs, w1p):
    ns, b, n = zs.shape
    hid = w1p.shape[2]
    bk = _blk(n, 1024)
    nk = n // bk
    return pl.pallas_call(
        functools.partial(_mlp1_kernel, ns=ns, nk=nk),
        grid=(ns, nk),
        in_specs=[
            pl.BlockSpec((1, b, bk), lambda s, k: (s, 0, k)),
            pl.BlockSpec((1, bk, hid), lambda s, k: (s, k, 0)),
        ],
        out_specs=pl.BlockSpec((b, hid), lambda s, k: (0, 0)),
        out_shape=jax.ShapeDtypeStruct((b, hid), jnp.float32),
        scratch_shapes=[pltpu.VMEM((b, hid), jnp.float32)],
        compiler_params=pltpu.CompilerParams(
            dimension_semantics=("arbitrary", "arbitrary")),
    )(zs, w1p)


def _mlp2_kernel(h_ref, b1_ref, w1l_ref, tf_ref, w2_ref, b2_ref, x_ref,
                 o_ref, *, inv_b):
    j = pl.program_id(0)
    h = h_ref[...] + b1_ref[...] + tf_ref[...] * w1l_ref[...]
    hs = (h * jax.nn.sigmoid(h)).astype(jnp.bfloat16)
    pred = jax.lax.dot_general(
        hs, w2_ref[...], (((1,), (0,)), ((), ())),
        preferred_element_type=jnp.float32) + b2_ref[...]
    d = x_ref[...] - pred
    part = jnp.sum(d * d) * inv_b

    @pl.when(j == 0)
    def _():
        o_ref[...] = jnp.zeros_like(o_ref)

    o_ref[...] = o_ref[...] + part


def _mlp2_loss(h, b1_2d, w1_last, tf, w2, b2_2d, x):
    b, hid = h.shape
    n = w2.shape[1]
    bn = _blk(n, 1024)
    out = pl.pallas_call(
        functools.partial(_mlp2_kernel, inv_b=1.0 / b),
        grid=(n // bn,),
        in_specs=[
            pl.BlockSpec((b, hid), lambda j: (0, 0)),
            pl.BlockSpec((1, hid), lambda j: (0, 0)),
            pl.BlockSpec((1, hid), lambda j: (0, 0)),
            pl.BlockSpec((b, 1), lambda j: (0, 0)),
            pl.BlockSpec((hid, bn), lambda j: (0, j)),
            pl.BlockSpec((1, bn), lambda j: (0, j)),
            pl.BlockSpec((b, bn), lambda j: (0, j)),
        ],
        out_specs=pl.BlockSpec((1, 1), lambda j: (0, 0)),
        out_shape=jax.ShapeDtypeStruct((1, 1), jnp.float32),
        compiler_params=pltpu.CompilerParams(
            dimension_semantics=("arbitrary",)),
    )(h, b1_2d, w1_last, tf, w2, b2_2d, x)
    return out[0, 0]


def kernel(x, entity_emb, weight_velocity, W_second, b_second, W1, b1, W2, b2,
           adj_vals, eigen_val0, adj_rows, adj_cols, kg_src, kg_dst,
           batch_item_ids):
    b, n_items = x.shape
    n_ent, latdim = entity_emb.shape
    hid = W1.shape[1]
    np_ = _round_up(n_items, 128 if n_items < 1024 else 1024)
    pad = np_ - n_items

    key = jax.random.key(42)
    kt, kd = jax.random.split(key)
    t = jax.random.randint(kt, (b, 1), 1, _T + 1)
    tf = t.astype(jnp.float32)
    keep = jax.random.bernoulli(kd, 1.0 - _DROPOUT, x.shape)

    inv_keep = 1.0 / (1.0 - _DROPOUT)
    c = jnp.where(keep, inv_keep * x, 0.0)
    xp = jnp.pad(x, ((0, 0), (0, pad)))
    mbp = jnp.pad(jnp.concatenate([x, c], axis=0).astype(jnp.bfloat16),
                  ((0, 0), (0, pad)))

    a_dense = jnp.zeros((np_, np_), jnp.float32)
    a_dense = a_dense.at[adj_rows, adj_cols].add(adj_vals)

    p = _matmul(mbp, a_dense, trans_b=True, out_dtype=jnp.bfloat16)
    s = _matmul(p, a_dense, trans_b=False, out_dtype=jnp.float32)

    d1 = _matmul(entity_emb.astype(jnp.bfloat16), weight_velocity)
    src_p = kg_src.astype(jnp.int32)
    dst_p = kg_dst.astype(jnp.int32)
    inv = jnp.full((_round_up(n_ent + 1, 8),), -1, jnp.int32)
    inv = inv.at[batch_item_ids].set(jnp.arange(b, dtype=jnp.int32))
    out_sc = _kg_divergence_sc(d1, entity_emb, src_p, dst_p, inv)
    acc_tot = out_sc[0, :b] + out_sc[1, :b]
    secg = acc_tot[inv[batch_item_ids]]

    gamma = _ALPHA * tf / _T
    inv_eig = (1.0 / eigen_val0).reshape(1, 1)
    wsp = jnp.pad(W_second.astype(jnp.bfloat16), ((0, 0), (0, pad)))
    bsp = jnp.pad(b_second.reshape(1, n_items), ((0, 0), (0, pad)))
    zs = _assemble(xp, mbp, s, secg, wsp, bsp, 1.0 - gamma,
                   gamma * inv_eig[0, 0], inv_eig)

    w1p = _w1_pack(W1, n_items, np_, hid)
    h = _mlp1(zs, w1p)

    w2p = jnp.pad(W2.astype(jnp.bfloat16), ((0, 0), (0, pad)))
    b2p = jnp.pad(b2.reshape(1, n_items), ((0, 0), (0, pad)))
    loss = _mlp2_loss(h, b1.reshape(1, -1), W1[-1:, :], tf, w2p, b2p, xp)
    return loss


# speedup vs baseline: 6.5638x; 1.0191x over previous
import functools

import jax
import jax.numpy as jnp
from jax import lax
from jax.experimental import pallas as pl
from jax.experimental.pallas import tpu as pltpu
from jax.experimental.pallas import tpu_sc as plsc

_ALPHA = 2.5
_T = 2
_DROPOUT = 0.5


def _round_up(n, m):
    return ((n + m - 1) // m) * m


def _blk(n, target):
    b = min(n, target)
    while n % b:
        b -= 128
    return b


def _mm_kernel(a_ref, b_ref, o_ref, acc_ref, *, nk, trans_b):
    k = pl.program_id(2)

    @pl.when(k == 0)
    def _():
        acc_ref[...] = jnp.zeros_like(acc_ref)

    dn = (((1,), (1,)), ((), ())) if trans_b else (((1,), (0,)), ((), ()))
    acc_ref[...] += jax.lax.dot_general(
        a_ref[...], b_ref[...].astype(jnp.bfloat16), dn,
        preferred_element_type=jnp.float32)

    @pl.when(k == nk - 1)
    def _():
        o_ref[...] = acc_ref[...].astype(o_ref.dtype)


def _matmul(a, b, *, trans_b=False, out_dtype=jnp.float32, bm=None, bn=None,
            bk=None):
    m, ka = a.shape
    if trans_b:
        n, kb = b.shape
    else:
        kb, n = b.shape
    assert ka == kb
    bm = bm or m
    bn = bn or _blk(n, 1024)
    bk = bk or _blk(ka, 1024)
    nk = ka // bk
    b_spec = (pl.BlockSpec((bn, bk), lambda i, j, k: (j, k)) if trans_b
              else pl.BlockSpec((bk, bn), lambda i, j, k: (k, j)))
    return pl.pallas_call(
        functools.partial(_mm_kernel, nk=nk, trans_b=trans_b),
        grid=(m // bm, n // bn, nk),
        in_specs=[pl.BlockSpec((bm, bk), lambda i, j, k: (i, k)), b_spec],
        out_specs=pl.BlockSpec((bm, bn), lambda i, j, k: (i, j)),
        out_shape=jax.ShapeDtypeStruct((m, n), out_dtype),
        scratch_shapes=[pltpu.VMEM((bm, bn), jnp.float32)],
        compiler_params=pltpu.CompilerParams(
            dimension_semantics=("parallel", "parallel", "arbitrary")),
    )(a, b)


_NC, _NS = 2, 16
_EPW = 5120
_ACC_ROWS = _NS * 72


def _kg_sc_body(d1_hbm, emb_hbm, src_hbm, dst_hbm, inv_hbm, out_hbm,
                inv_v, srcb, dstb, ks1d, kd1d, km1d, m2d,
                g_d1d, g_d1s, g_emb, msg_v, acc_sh, sem1, sem2, sem3,
                *, n_ent, e_per_w):
    c = lax.axis_index("c")
    s = lax.axis_index("s")
    wid = c * _NS + s

    def zrow(r, _):
        for l in range(8):
            msg_v[r, pl.ds(l * 16, 16)] = jnp.zeros((16,), jnp.float32)
        return 0

    lax.fori_loop(0, 128, zrow, 0)
    pltpu.sync_copy(msg_v.at[pl.ds(0, 72)], acc_sh.at[pl.ds(s * 72, 72)])
    plsc.subcore_barrier()

    def pre(i, _):
        z = jnp.zeros((16,), jnp.int32)
        srcb[pl.ds(i * 16, 16)] = z + n_ent
        dstb[pl.ds(i * 16, 16)] = z
        ks1d[pl.ds(i * 16, 16)] = z
        kd1d[pl.ds(i * 16, 16)] = z
        km1d[pl.ds(i * 16, 16)] = z + (_ACC_ROWS - 16)
        return 0

    lax.fori_loop(0, _EPW // 16, pre, 0)

    pltpu.sync_copy(inv_hbm, inv_v)
    pltpu.sync_copy(src_hbm.at[pl.ds(wid * e_per_w, e_per_w)],
                    srcb.at[pl.ds(0, e_per_w)])
    pltpu.sync_copy(dst_hbm.at[pl.ds(wid * e_per_w, e_per_w)],
                    dstb.at[pl.ds(0, e_per_w)])

    def comp(i, off):
        sv = srcb[pl.ds(i * 16, 16)]
        dv = dstb[pl.ds(i * 16, 16)]
        mv = plsc.load_gather(inv_v, [sv])
        mask = mv >= 0
        plsc.store_compressed(ks1d.at[pl.ds(off, 16)], sv, mask=mask)
        plsc.store_compressed(kd1d.at[pl.ds(off, 16)], dv, mask=mask)
        plsc.store_compressed(km1d.at[pl.ds(off, 16)], mv, mask=mask)
        return off + jnp.sum(mask.astype(jnp.int32))

    kcount = lax.fori_loop(0, _EPW // 16, comp, jnp.int32(0))

    def relay(i, _):
        r = i // 8
        ll = (i % 8) * 16
        m2d[r, pl.ds(ll, 16)] = km1d[pl.ds(i * 16, 16)]
        return 0

    lax.fori_loop(0, _EPW // 16, relay, 0)

    def chunk(ch, _):
        i0 = ch * 128
        cp1 = pltpu.async_copy(d1_hbm.at[kd1d.at[pl.ds(i0, 128)]], g_d1d, sem1)
        cp2 = pltpu.async_copy(d1_hbm.at[ks1d.at[pl.ds(i0, 128)]], g_d1s, sem2)
        cp3 = pltpu.async_copy(emb_hbm.at[kd1d.at[pl.ds(i0, 128)]], g_emb, sem3)
        cp1.wait()
        cp2.wait()
        cp3.wait()

        def mrow(r, _):
            for l in range(8):
                sl = pl.ds(l * 16, 16)
                x2 = 2.0 * (g_d1d[r, sl] - g_d1s[r, sl])
                th = 1.0 - 2.0 / (jnp.exp(x2) + 1.0)
                msg_v[r, sl] = th * g_emb[r, sl]
            return 0

        lax.fori_loop(0, 128, mrow, 0)
        pltpu.sync_copy(msg_v, acc_sh.at[m2d.at[ch]], add=True)
        return 0

    nch = (kcount + 127) // 128
    lax.fori_loop(0, nch, chunk, 0)

    plsc.subcore_barrier()
    pltpu.sync_copy(acc_sh.at[pl.ds(s * 72, 72)],
                    out_hbm.at[c, pl.ds(s * 72, 72)])


def _kg_divergence_sc(d1, emb, src_p, dst_p, inv):
    mesh = plsc.VectorSubcoreMesh(core_axis_name="c", subcore_axis_name="s",
                                  num_cores=_NC, num_subcores=_NS)
    latdim = emb.shape[1]
    f32 = jnp.float32
    e = src_p.shape[0]
    e_per_w = e // (_NC * _NS)
    assert e_per_w % 8 == 0 and e_per_w <= _EPW
    return pl.kernel(
        functools.partial(_kg_sc_body, n_ent=emb.shape[0], e_per_w=e_per_w),
        out_type=jax.ShapeDtypeStruct((_NC, _ACC_ROWS, latdim), f32),
        mesh=mesh,
        scratch_types=[
            pltpu.VMEM(inv.shape, jnp.int32),
            pltpu.VMEM((_EPW,), jnp.int32),
            pltpu.VMEM((_EPW,), jnp.int32),
            pltpu.VMEM((_EPW,), jnp.int32),
            pltpu.VMEM((_EPW,), jnp.int32),
            pltpu.VMEM((_EPW,), jnp.int32),
            pltpu.VMEM((40, 128), jnp.int32),
            pltpu.VMEM((128, latdim), f32),
            pltpu.VMEM((128, latdim), f32),
            pltpu.VMEM((128, latdim), f32),
            pltpu.VMEM((128, latdim), f32),
            pltpu.VMEM_SHARED((_ACC_ROWS, latdim), f32),
            pltpu.SemaphoreType.DMA,
            pltpu.SemaphoreType.DMA,
            pltpu.SemaphoreType.DMA,
        ],
        compiler_params=pltpu.CompilerParams(needs_layout_passes=False),
    )(d1, emb, src_p, dst_p, inv)


def _assemble_kernel(x_ref, c_ref, ax_ref, ac_ref, secg_ref, ws_ref, bs_ref,
                     g1_ref, g2_ref, inv_eig_ref, o_ref):
    x = x_ref[...]
    sec = jax.lax.dot_general(
        secg_ref[...], ws_ref[...], (((1,), (0,)), ((), ())),
        preferred_element_type=jnp.float32) + bs_ref[...]
    z = x * g1_ref[...] + g2_ref[...] * ax_ref[...] + sec
    o_ref[0, :, :] = z.astype(o_ref.dtype)
    o_ref[1, :, :] = c_ref[...].astype(o_ref.dtype)
    o_ref[2, :, :] = (ac_ref[...] * inv_eig_ref[...]).astype(o_ref.dtype)


def _assemble(x, c_bf, s, secg, w_second, b_second2d, g1, g2, inv_eig):
    b, n = x.shape
    d = secg.shape[1]
    bn = _blk(n, 1024)
    return pl.pallas_call(
        _assemble_kernel,
        grid=(n // bn,),
        in_specs=[
            pl.BlockSpec((b, bn), lambda j: (0, j)),
            pl.BlockSpec((b, bn), lambda j: (1, j)),
            pl.BlockSpec((b, bn), lambda j: (0, j)),
            pl.BlockSpec((b, bn), lambda j: (1, j)),
            pl.BlockSpec((b, d), lambda j: (0, 0)),
            pl.BlockSpec((d, bn), lambda j: (0, j)),
            pl.BlockSpec((1, bn), lambda j: (0, j)),
            pl.BlockSpec((b, 1), lambda j: (0, 0)),
            pl.BlockSpec((b, 1), lambda j: (0, 0)),
            pl.BlockSpec((1, 1), lambda j: (0, 0)),
        ],
        out_specs=pl.BlockSpec((3, b, bn), lambda j: (0, 0, j)),
        out_shape=jax.ShapeDtypeStruct((3, b, n), jnp.bfloat16),
        compiler_params=pltpu.CompilerParams(
            dimension_semantics=("arbitrary",)),
    )(x, c_bf, s, s, secg, w_second, b_second2d, g1, g2, inv_eig)


def _w1_pack_kernel(w1_ref, o_ref, *, nk):
    k = pl.program_id(1)
    blk = w1_ref[...].astype(jnp.bfloat16)
    o_ref[0] = jnp.where(k < nk, blk, jnp.zeros_like(blk))


def _w1_pack(w1, n_items, np_, hid, bk=1000):
    bk = min(bk, n_items)
    nk = n_items // bk
    nk_pad = -(-np_ // bk)

    def idx_in(s, k):
        kk = jnp.minimum(k, nk - 1)
        return (s * nk + kk, 0)

    return pl.pallas_call(
        functools.partial(_w1_pack_kernel, nk=nk),
        grid=(3, nk_pad),
        in_specs=[pl.BlockSpec((bk, hid), idx_in)],
        out_specs=pl.BlockSpec((1, bk, hid), lambda s, k: (s, k, 0)),
        out_shape=jax.ShapeDtypeStruct((3, np_, hid), jnp.bfloat16),
        compiler_params=pltpu.CompilerParams(
            dimension_semantics=("arbitrary", "arbitrary")),
    )(w1)


def _mlp1_kernel(z_ref, w1_ref, o_ref, acc_ref, *, ns, nk):
    s = pl.program_id(0)
    k = pl.program_id(1)

    @pl.when((s == 0) & (k == 0))
    def _():
        acc_ref[...] = jnp.zeros_like(acc_ref)

    acc_ref[...] += jax.lax.dot_general(
        z_ref[0], w1_ref[0], (((1,), (0,)), ((), ())),
        preferred_element_type=jnp.float32)

    @pl.when((s == ns - 1) & (k == nk - 1))
    def _():
        o_ref[...] = acc_ref[...]


def _mlp1(zs, w1p):
    ns, b, n = zs.shape
    hid = w1p.shape[2]
    bk = _blk(n, 1024)
    nk = n // bk
    return pl.pallas_call(
        functools.partial(_mlp1_kernel, ns=ns, nk=nk),
        grid=(ns, nk),
        in_specs=[
            pl.BlockSpec((1, b, bk), lambda s, k: (s, 0, k)),
            pl.BlockSpec((1, bk, hid), lambda s, k: (s, k, 0)),
        ],
        out_specs=pl.BlockSpec((b, hid), lambda s, k: (0, 0)),
        out_shape=jax.ShapeDtypeStruct((b, hid), jnp.float32),
        scratch_shapes=[pltpu.VMEM((b, hid), jnp.float32)],
        compiler_params=pltpu.CompilerParams(
            dimension_semantics=("arbitrary", "arbitrary")),
    )(zs, w1p)


def _mlp2_kernel(h_ref, b1_ref, w1l_ref, tf_ref, w2_ref, b2_ref, x_ref,
                 o_ref, *, inv_b):
    j = pl.program_id(0)
    h = h_ref[...] + b1_ref[...] + tf_ref[...] * w1l_ref[...]
    hs = (h * jax.nn.sigmoid(h)).astype(jnp.bfloat16)
    pred = jax.lax.dot_general(
        hs, w2_ref[...], (((1,), (0,)), ((), ())),
        preferred_element_type=jnp.float32) + b2_ref[...]
    d = x_ref[...] - pred
    part = jnp.sum(d * d) * inv_b

    @pl.when(j == 0)
    def _():
        o_ref[...] = jnp.zeros_like(o_ref)

    o_ref[...] = o_ref[...] + part


def _mlp2_loss(h, b1_2d, w1_last, tf, w2, b2_2d, x):
    b, hid = h.shape
    n = w2.shape[1]
    bn = _blk(n, 1024)
    out = pl.pallas_call(
        functools.partial(_mlp2_kernel, inv_b=1.0 / b),
        grid=(n // bn,),
        in_specs=[
            pl.BlockSpec((b, hid), lambda j: (0, 0)),
            pl.BlockSpec((1, hid), lambda j: (0, 0)),
            pl.BlockSpec((1, hid), lambda j: (0, 0)),
            pl.BlockSpec((b, 1), lambda j: (0, 0)),
            pl.BlockSpec((hid, bn), lambda j: (0, j)),
            pl.BlockSpec((1, bn), lambda j: (0, j)),
            pl.BlockSpec((b, bn), lambda j: (0, j)),
        ],
        out_specs=pl.BlockSpec((1, 1), lambda j: (0, 0)),
        out_shape=jax.ShapeDtypeStruct((1, 1), jnp.float32),
        compiler_params=pltpu.CompilerParams(
            dimension_semantics=("arbitrary",)),
    )(h, b1_2d, w1_last, tf, w2, b2_2d, x)
    return out[0, 0]


def kernel(x, entity_emb, weight_velocity, W_second, b_second, W1, b1, W2, b2,
           adj_vals, eigen_val0, adj_rows, adj_cols, kg_src, kg_dst,
           batch_item_ids):
    b, n_items = x.shape
    n_ent, latdim = entity_emb.shape
    hid = W1.shape[1]
    np_ = _round_up(n_items, 128 if n_items < 1024 else 1024)
    pad = np_ - n_items

    key = jax.random.key(42)
    kt, kd = jax.random.split(key)
    t = jax.random.randint(kt, (b, 1), 1, _T + 1)
    tf = t.astype(jnp.float32)
    keep = jax.random.bernoulli(kd, 1.0 - _DROPOUT, x.shape)

    inv_keep = 1.0 / (1.0 - _DROPOUT)
    c = jnp.where(keep, inv_keep * x, 0.0)
    xp = jnp.pad(x, ((0, 0), (0, pad)))
    mbp = jnp.pad(jnp.concatenate([x, c], axis=0).astype(jnp.bfloat16),
                  ((0, 0), (0, pad)))

    flat_idx = adj_rows * np_ + adj_cols
    a_dense = jnp.zeros((np_ * np_,), jnp.float32)
    a_dense = a_dense.at[flat_idx].add(adj_vals).reshape(np_, np_)

    p = _matmul(mbp, a_dense, trans_b=True, out_dtype=jnp.bfloat16)
    s = _matmul(p, a_dense, trans_b=False, out_dtype=jnp.float32)

    d1 = _matmul(entity_emb.astype(jnp.bfloat16), weight_velocity)
    src_p = kg_src.astype(jnp.int32)
    dst_p = kg_dst.astype(jnp.int32)
    inv = jnp.full((_round_up(n_ent + 1, 8),), -1, jnp.int32)
    inv = inv.at[batch_item_ids].set(jnp.arange(b, dtype=jnp.int32))
    out_sc = _kg_divergence_sc(d1, entity_emb, src_p, dst_p, inv)
    acc_tot = out_sc[0, :b] + out_sc[1, :b]
    secg = acc_tot[inv[batch_item_ids]]

    gamma = _ALPHA * tf / _T
    inv_eig = (1.0 / eigen_val0).reshape(1, 1)
    wsp = jnp.pad(W_second.astype(jnp.bfloat16), ((0, 0), (0, pad)))
    bsp = jnp.pad(b_second.reshape(1, n_items), ((0, 0), (0, pad)))
    zs = _assemble(xp, mbp, s, secg, wsp, bsp, 1.0 - gamma,
                   gamma * inv_eig[0, 0], inv_eig)

    w1p = _w1_pack(W1, n_items, np_, hid)
    h = _mlp1(zs, w1p)

    w2p = jnp.pad(W2.astype(jnp.bfloat16), ((0, 0), (0, pad)))
    b2p = jnp.pad(b2.reshape(1, n_items), ((0, 0), (0, pad)))
    loss = _mlp2_loss(h, b1.reshape(1, -1), W1[-1:, :], tf, w2p, b2p, xp)
    return loss


# speedup vs baseline: 6.6233x; 1.0091x over previous
import functools

import jax
import jax.numpy as jnp
from jax import lax
from jax.experimental import pallas as pl
from jax.experimental.pallas import tpu as pltpu
from jax.experimental.pallas import tpu_sc as plsc

_ALPHA = 2.5
_T = 2
_DROPOUT = 0.5


def _round_up(n, m):
    return ((n + m - 1) // m) * m


def _blk(n, target):
    b = min(n, target)
    while n % b:
        b -= 128
    return b


def _mm_kernel(a_ref, b_ref, o_ref, acc_ref, *, nk, trans_b):
    k = pl.program_id(2)

    @pl.when(k == 0)
    def _():
        acc_ref[...] = jnp.zeros_like(acc_ref)

    dn = (((1,), (1,)), ((), ())) if trans_b else (((1,), (0,)), ((), ()))
    acc_ref[...] += jax.lax.dot_general(
        a_ref[...], b_ref[...].astype(jnp.bfloat16), dn,
        preferred_element_type=jnp.float32)

    @pl.when(k == nk - 1)
    def _():
        o_ref[...] = acc_ref[...].astype(o_ref.dtype)


def _matmul(a, b, *, trans_b=False, out_dtype=jnp.float32, bm=None, bn=None,
            bk=None):
    m, ka = a.shape
    if trans_b:
        n, kb = b.shape
    else:
        kb, n = b.shape
    assert ka == kb
    bm = bm or m
    bn = bn or _blk(n, 1024)
    bk = bk or _blk(ka, 1024)
    nk = ka // bk
    b_spec = (pl.BlockSpec((bn, bk), lambda i, j, k: (j, k)) if trans_b
              else pl.BlockSpec((bk, bn), lambda i, j, k: (k, j)))
    return pl.pallas_call(
        functools.partial(_mm_kernel, nk=nk, trans_b=trans_b),
        grid=(m // bm, n // bn, nk),
        in_specs=[pl.BlockSpec((bm, bk), lambda i, j, k: (i, k)), b_spec],
        out_specs=pl.BlockSpec((bm, bn), lambda i, j, k: (i, j)),
        out_shape=jax.ShapeDtypeStruct((m, n), out_dtype),
        scratch_shapes=[pltpu.VMEM((bm, bn), jnp.float32)],
        compiler_params=pltpu.CompilerParams(
            dimension_semantics=("parallel", "parallel", "arbitrary")),
    )(a, b)


_NC, _NS = 2, 16
_EPW = 5120
_ACC_ROWS = _NS * 72


def _kg_sc_body(d1_hbm, emb_hbm, src_hbm, dst_hbm, inv_hbm, out_hbm,
                inv_v, srcb, dstb, ks1d, kd1d, km1d, m2d,
                g_d1d, g_d1s, g_emb, msg_v, acc_sh, sem1, sem2, sem3,
                *, n_ent, e_per_w):
    c = lax.axis_index("c")
    s = lax.axis_index("s")
    wid = c * _NS + s

    def zrow(r, _):
        for l in range(8):
            msg_v[r, pl.ds(l * 16, 16)] = jnp.zeros((16,), jnp.float32)
        return 0

    lax.fori_loop(0, 128, zrow, 0)
    pltpu.sync_copy(msg_v.at[pl.ds(0, 72)], acc_sh.at[pl.ds(s * 72, 72)])
    plsc.subcore_barrier()

    def pre(i, _):
        z = jnp.zeros((16,), jnp.int32)
        srcb[pl.ds(i * 16, 16)] = z + n_ent
        dstb[pl.ds(i * 16, 16)] = z
        ks1d[pl.ds(i * 16, 16)] = z
        kd1d[pl.ds(i * 16, 16)] = z
        km1d[pl.ds(i * 16, 16)] = z + (_ACC_ROWS - 16)
        return 0

    lax.fori_loop(0, _EPW // 16, pre, 0)

    pltpu.sync_copy(inv_hbm, inv_v)
    pltpu.sync_copy(src_hbm.at[pl.ds(wid * e_per_w, e_per_w)],
                    srcb.at[pl.ds(0, e_per_w)])
    pltpu.sync_copy(dst_hbm.at[pl.ds(wid * e_per_w, e_per_w)],
                    dstb.at[pl.ds(0, e_per_w)])

    def comp(i, off):
        sv = srcb[pl.ds(i * 16, 16)]
        dv = dstb[pl.ds(i * 16, 16)]
        mv = plsc.load_gather(inv_v, [sv])
        mask = mv >= 0
        plsc.store_compressed(ks1d.at[pl.ds(off, 16)], sv, mask=mask)
        plsc.store_compressed(kd1d.at[pl.ds(off, 16)], dv, mask=mask)
        plsc.store_compressed(km1d.at[pl.ds(off, 16)], mv, mask=mask)
        return off + jnp.sum(mask.astype(jnp.int32))

    kcount = lax.fori_loop(0, _EPW // 16, comp, jnp.int32(0))

    def relay(i, _):
        r = i // 8
        ll = (i % 8) * 16
        m2d[r, pl.ds(ll, 16)] = km1d[pl.ds(i * 16, 16)]
        return 0

    lax.fori_loop(0, _EPW // 16, relay, 0)

    def chunk(ch, _):
        i0 = ch * 128
        cp1 = pltpu.async_copy(d1_hbm.at[kd1d.at[pl.ds(i0, 128)]], g_d1d, sem1)
        cp2 = pltpu.async_copy(d1_hbm.at[ks1d.at[pl.ds(i0, 128)]], g_d1s, sem2)
        cp3 = pltpu.async_copy(emb_hbm.at[kd1d.at[pl.ds(i0, 128)]], g_emb, sem3)
        cp1.wait()
        cp2.wait()
        cp3.wait()

        def mrow(r, _):
            for l in range(8):
                sl = pl.ds(l * 16, 16)
                x2 = 2.0 * (g_d1d[r, sl] - g_d1s[r, sl])
                th = 1.0 - 2.0 / (jnp.exp(x2) + 1.0)
                msg_v[r, sl] = th * g_emb[r, sl]
            return 0

        lax.fori_loop(0, 128, mrow, 0)
        pltpu.sync_copy(msg_v, acc_sh.at[m2d.at[ch]], add=True)
        return 0

    nch = (kcount + 127) // 128
    lax.fori_loop(0, nch, chunk, 0)

    plsc.subcore_barrier()
    pltpu.sync_copy(acc_sh.at[pl.ds(s * 72, 72)],
                    out_hbm.at[c, pl.ds(s * 72, 72)])


def _kg_divergence_sc(d1, emb, src_p, dst_p, inv):
    mesh = plsc.VectorSubcoreMesh(core_axis_name="c", subcore_axis_name="s",
                                  num_cores=_NC, num_subcores=_NS)
    latdim = emb.shape[1]
    f32 = jnp.float32
    e = src_p.shape[0]
    e_per_w = e // (_NC * _NS)
    assert e_per_w % 8 == 0 and e_per_w <= _EPW
    return pl.kernel(
        functools.partial(_kg_sc_body, n_ent=emb.shape[0], e_per_w=e_per_w),
        out_type=jax.ShapeDtypeStruct((_NC, _ACC_ROWS, latdim), f32),
        mesh=mesh,
        scratch_types=[
            pltpu.VMEM(inv.shape, jnp.int32),
            pltpu.VMEM((_EPW,), jnp.int32),
            pltpu.VMEM((_EPW,), jnp.int32),
            pltpu.VMEM((_EPW,), jnp.int32),
            pltpu.VMEM((_EPW,), jnp.int32),
            pltpu.VMEM((_EPW,), jnp.int32),
            pltpu.VMEM((40, 128), jnp.int32),
            pltpu.VMEM((128, latdim), f32),
            pltpu.VMEM((128, latdim), f32),
            pltpu.VMEM((128, latdim), f32),
            pltpu.VMEM((128, latdim), f32),
            pltpu.VMEM_SHARED((_ACC_ROWS, latdim), f32),
            pltpu.SemaphoreType.DMA,
            pltpu.SemaphoreType.DMA,
            pltpu.SemaphoreType.DMA,
        ],
        compiler_params=pltpu.CompilerParams(needs_layout_passes=False),
    )(d1, emb, src_p, dst_p, inv)


def _assemble_kernel(x_ref, c_ref, ax_ref, ac_ref, secg_ref, ws_ref, bs_ref,
                     g1_ref, g2_ref, inv_eig_ref, o_ref):
    x = x_ref[...]
    sec = jax.lax.dot_general(
        secg_ref[...], ws_ref[...], (((1,), (0,)), ((), ())),
        preferred_element_type=jnp.float32) + bs_ref[...]
    z = x * g1_ref[...] + g2_ref[...] * ax_ref[...] + sec
    o_ref[0, :, :] = z.astype(o_ref.dtype)
    o_ref[1, :, :] = c_ref[...].astype(o_ref.dtype)
    o_ref[2, :, :] = (ac_ref[...] * inv_eig_ref[...]).astype(o_ref.dtype)


def _assemble(x, c_bf, s, secg, w_second, b_second2d, g1, g2, inv_eig):
    b, n = x.shape
    d = secg.shape[1]
    bn = _blk(n, 1024)
    return pl.pallas_call(
        _assemble_kernel,
        grid=(n // bn,),
        in_specs=[
            pl.BlockSpec((b, bn), lambda j: (0, j)),
            pl.BlockSpec((b, bn), lambda j: (1, j)),
            pl.BlockSpec((b, bn), lambda j: (0, j)),
            pl.BlockSpec((b, bn), lambda j: (1, j)),
            pl.BlockSpec((b, d), lambda j: (0, 0)),
            pl.BlockSpec((d, bn), lambda j: (0, j)),
            pl.BlockSpec((1, bn), lambda j: (0, j)),
            pl.BlockSpec((b, 1), lambda j: (0, 0)),
            pl.BlockSpec((b, 1), lambda j: (0, 0)),
            pl.BlockSpec((1, 1), lambda j: (0, 0)),
        ],
        out_specs=pl.BlockSpec((3, b, bn), lambda j: (0, 0, j)),
        out_shape=jax.ShapeDtypeStruct((3, b, n), jnp.bfloat16),
        compiler_params=pltpu.CompilerParams(
            dimension_semantics=("arbitrary",)),
    )(x, c_bf, s, s, secg, w_second, b_second2d, g1, g2, inv_eig)


def _w1_pack_kernel(w1_ref, o_ref, *, nk):
    k = pl.program_id(1)
    blk = w1_ref[...].astype(jnp.bfloat16)
    o_ref[0] = jnp.where(k < nk, blk, jnp.zeros_like(blk))


def _w1_pack(w1, n_items, np_, hid, bk=1000):
    bk = min(bk, n_items)
    nk = n_items // bk
    nk_pad = -(-np_ // bk)

    def idx_in(s, k):
        kk = jnp.minimum(k, nk - 1)
        return (s * nk + kk, 0)

    return pl.pallas_call(
        functools.partial(_w1_pack_kernel, nk=nk),
        grid=(3, nk_pad),
        in_specs=[pl.BlockSpec((bk, hid), idx_in)],
        out_specs=pl.BlockSpec((1, bk, hid), lambda s, k: (s, k, 0)),
        out_shape=jax.ShapeDtypeStruct((3, np_, hid), jnp.bfloat16),
        compiler_params=pltpu.CompilerParams(
            dimension_semantics=("arbitrary", "arbitrary")),
    )(w1)


def _mlp1_kernel(z_ref, w1_ref, o_ref, acc_ref, *, ns, nk):
    s = pl.program_id(0)
    k = pl.program_id(1)

    @pl.when((s == 0) & (k == 0))
    def _():
        acc_ref[...] = jnp.zeros_like(acc_ref)

    acc_ref[...] += jax.lax.dot_general(
        z_ref[0], w1_ref[0], (((1,), (0,)), ((), ())),
        preferred_element_type=jnp.float32)

    @pl.when((s == ns - 1) & (k == nk - 1))
    def _():
        o_ref[...] = acc_ref[...]


def _mlp1(zs, w1p):
    ns, b, n = zs.shape
    hid = w1p.shape[2]
    bk = _blk(n, 1024)
    nk = n // bk
    return pl.pallas_call(
        functools.partial(_mlp1_kernel, ns=ns, nk=nk),
        grid=(ns, nk),
        in_specs=[
            pl.BlockSpec((1, b, bk), lambda s, k: (s, 0, k)),
            pl.BlockSpec((1, bk, hid), lambda s, k: (s, k, 0)),
        ],
        out_specs=pl.BlockSpec((b, hid), lambda s, k: (0, 0)),
        out_shape=jax.ShapeDtypeStruct((b, hid), jnp.float32),
        scratch_shapes=[pltpu.VMEM((b, hid), jnp.float32)],
        compiler_params=pltpu.CompilerParams(
            dimension_semantics=("arbitrary", "arbitrary")),
    )(zs, w1p)


def _mlp2_kernel(h_ref, b1_ref, w1l_ref, tf_ref, w2_ref, b2_ref, x_ref,
                 o_ref, *, inv_b):
    j = pl.program_id(0)
    h = h_ref[...] + b1_ref[...] + tf_ref[...] * w1l_ref[...]
    hs = (h * jax.nn.sigmoid(h)).astype(jnp.bfloat16)
    pred = jax.lax.dot_general(
        hs, w2_ref[...], (((1,), (0,)), ((), ())),
        preferred_element_type=jnp.float32) + b2_ref[...]
    d = x_ref[...] - pred
    part = jnp.sum(d * d) * inv_b

    @pl.when(j == 0)
    def _():
        o_ref[...] = jnp.zeros_like(o_ref)

    o_ref[...] = o_ref[...] + part


def _mlp2_loss(h, b1_2d, w1_last, tf, w2, b2_2d, x):
    b, hid = h.shape
    n = w2.shape[1]
    bn = _blk(n, 1024)
    out = pl.pallas_call(
        functools.partial(_mlp2_kernel, inv_b=1.0 / b),
        grid=(n // bn,),
        in_specs=[
            pl.BlockSpec((b, hid), lambda j: (0, 0)),
            pl.BlockSpec((1, hid), lambda j: (0, 0)),
            pl.BlockSpec((1, hid), lambda j: (0, 0)),
            pl.BlockSpec((b, 1), lambda j: (0, 0)),
            pl.BlockSpec((hid, bn), lambda j: (0, j)),
            pl.BlockSpec((1, bn), lambda j: (0, j)),
            pl.BlockSpec((b, bn), lambda j: (0, j)),
        ],
        out_specs=pl.BlockSpec((1, 1), lambda j: (0, 0)),
        out_shape=jax.ShapeDtypeStruct((1, 1), jnp.float32),
        compiler_params=pltpu.CompilerParams(
            dimension_semantics=("arbitrary",)),
    )(h, b1_2d, w1_last, tf, w2, b2_2d, x)
    return out[0, 0]


def kernel(x, entity_emb, weight_velocity, W_second, b_second, W1, b1, W2, b2,
           adj_vals, eigen_val0, adj_rows, adj_cols, kg_src, kg_dst,
           batch_item_ids):
    b, n_items = x.shape
    n_ent, latdim = entity_emb.shape
    hid = W1.shape[1]
    np_ = _round_up(n_items, 128 if n_items < 1024 else 1024)
    pad = np_ - n_items

    key = jax.random.key(42)
    kt, kd = jax.random.split(key)
    t = jax.random.randint(kt, (b, 1), 1, _T + 1)
    tf = t.astype(jnp.float32)
    keep = jax.random.bernoulli(kd, 1.0 - _DROPOUT, x.shape)

    inv_keep = 1.0 / (1.0 - _DROPOUT)
    c = jnp.where(keep, inv_keep * x, 0.0)
    xp = jnp.pad(x, ((0, 0), (0, pad)))
    mbp = jnp.pad(jnp.concatenate([x, c], axis=0).astype(jnp.bfloat16),
                  ((0, 0), (0, pad)))

    flat_idx = adj_rows * np_ + adj_cols
    a_dense = jnp.zeros((np_ * np_,), jnp.float32)
    a_dense = a_dense.at[flat_idx].add(adj_vals).reshape(np_, np_)

    p = _matmul(mbp, a_dense, trans_b=True, out_dtype=jnp.bfloat16, bn=2048,
                bk=512)
    s = _matmul(p, a_dense, trans_b=False, out_dtype=jnp.bfloat16, bn=2048,
                bk=512)

    d1 = _matmul(entity_emb.astype(jnp.bfloat16), weight_velocity)
    src_p = kg_src.astype(jnp.int32)
    dst_p = kg_dst.astype(jnp.int32)
    inv = jnp.full((_round_up(n_ent + 1, 8),), -1, jnp.int32)
    inv = inv.at[batch_item_ids].set(jnp.arange(b, dtype=jnp.int32))
    out_sc = _kg_divergence_sc(d1, entity_emb, src_p, dst_p, inv)
    acc_tot = out_sc[0, :b] + out_sc[1, :b]
    secg = acc_tot[inv[batch_item_ids]]

    gamma = _ALPHA * tf / _T
    inv_eig = (1.0 / eigen_val0).reshape(1, 1)
    wsp = jnp.pad(W_second.astype(jnp.bfloat16), ((0, 0), (0, pad)))
    bsp = jnp.pad(b_second.reshape(1, n_items), ((0, 0), (0, pad)))
    zs = _assemble(xp, mbp, s, secg, wsp, bsp, 1.0 - gamma,
                   gamma * inv_eig[0, 0], inv_eig)

    w1p = _w1_pack(W1, n_items, np_, hid)
    h = _mlp1(zs, w1p)

    w2p = jnp.pad(W2.astype(jnp.bfloat16), ((0, 0), (0, pad)))
    b2p = jnp.pad(b2.reshape(1, n_items), ((0, 0), (0, pad)))
    loss = _mlp2_loss(h, b1.reshape(1, -1), W1[-1:, :], tf, w2p, b2p, xp)
    return loss
